# Initial kernel scaffold; baseline (speedup 1.0000x reference)
import jax
import jax.numpy as jnp
from jax.experimental import pallas as pl


def kernel(warped_xyz, f2_xyz, warped_points, f2_points, params):
    raise NotImplementedError("write your pallas kernel here")


# speedup vs baseline: 20.8308x; 20.8308x over previous
import functools

import jax
import jax.numpy as jnp
from jax import lax
from jax.experimental import pallas as pl
from jax.experimental.pallas import tpu as pltpu
from jax.experimental.pallas import tpu_sc as plsc

F32 = jnp.float32
_TN = 256
_K1 = 6
_K2 = 4
_TW = 128
_NW = 32
_CHUNK = 128


def _first_step():
    return (pl.program_id(0) == 0) & (pl.program_id(1) == 0)


def _knn_body(q_ref, xt_ref, idx_ref, *, k, m):
    q = q_ref[0]
    xt = xt_ref[0]
    d = -2.0 * jnp.dot(q, xt, preferred_element_type=F32)
    d = d + jnp.sum(q * q, axis=1, keepdims=True)
    d = d + jnp.sum(xt * xt, axis=0, keepdims=True)
    iota = lax.broadcasted_iota(jnp.int32, d.shape, 1)
    cols = []
    for _ in range(k):
        mn = jnp.min(d, axis=1, keepdims=True)
        am = jnp.min(jnp.where(d == mn, iota, m), axis=1, keepdims=True)
        cols.append(am)
        d = jnp.where(iota == am, jnp.inf, d)
    idx_ref[0] = jnp.concatenate(cols, axis=1) + pl.program_id(0) * m


def _knn_call(q8, xt8, k):
    b, n, _ = q8.shape
    m = xt8.shape[2]
    return pl.pallas_call(
        functools.partial(_knn_body, k=k, m=m),
        grid=(b, n // _TN),
        in_specs=[
            pl.BlockSpec((1, _TN, 8), lambda bb, i: (bb, i, 0)),
            pl.BlockSpec((1, 8, m), lambda bb, i: (bb, 0, 0)),
        ],
        out_specs=pl.BlockSpec((1, _TN, k), lambda bb, i: (bb, i, 0)),
        out_shape=jax.ShapeDtypeStruct((b, n, k), jnp.int32),
    )(q8, xt8)


def _gather_rows(table, idx):
    rows = idx.shape[0]
    per_w = rows // _NW
    chunks = per_w // _CHUNK
    idx3 = idx.reshape(_NW, chunks, _CHUNK)
    mesh = plsc.VectorSubcoreMesh(core_axis_name="c", subcore_axis_name="s",
                                  num_cores=2)

    @functools.partial(
        pl.kernel, mesh=mesh,
        out_type=jax.ShapeDtypeStruct((rows, _TW), F32),
        scratch_types=[
            pltpu.VMEM((chunks, _CHUNK), jnp.int32),
            pltpu.VMEM((2, _CHUNK, _TW), F32),
            pltpu.SemaphoreType.DMA,
            pltpu.SemaphoreType.DMA,
        ],
    )
    def gk(table_hbm, idx_hbm, out_hbm, idx_v, rows_v, sem0, sem1):
        wid = lax.axis_index("s") * 2 + lax.axis_index("c")
        base = wid * per_w
        pltpu.sync_copy(idx_hbm.at[wid], idx_v)
        sems = (sem0, sem1)
        prev = None
        for j in range(chunks):
            cp = pltpu.async_copy(table_hbm.at[idx_v.at[j]],
                                  rows_v.at[j % 2], sems[j % 2])
            if prev is not None:
                pj, pcp = prev
                pcp.wait()
                pltpu.sync_copy(
                    rows_v.at[pj % 2],
                    out_hbm.at[pl.ds(base + pj * _CHUNK, _CHUNK)])
            prev = (j, cp)
        pj, pcp = prev
        pcp.wait()
        pltpu.sync_copy(rows_v.at[pj % 2],
                        out_hbm.at[pl.ds(base + pj * _CHUNK, _CHUNK)])

    return gk(table, idx3)


def _p1_body(xyz_ref, wp_ref, g_ref, w1a_ref, w1b_ref, w1c_ref, we_ref,
             y1_ref, ye_ref, s1_ref, se_ref):
    @pl.when(_first_step())
    def _():
        s1_ref[...] = jnp.zeros_like(s1_ref)
        se_ref[...] = jnp.zeros_like(se_ref)

    q3 = xyz_ref[0][:, 0:3]
    wp = wp_ref[0]
    s1s = jnp.zeros((1, y1_ref.shape[-1]), F32)
    s1q = jnp.zeros((1, y1_ref.shape[-1]), F32)
    ses = jnp.zeros((1, ye_ref.shape[-1]), F32)
    seq = jnp.zeros((1, ye_ref.shape[-1]), F32)
    for j in range(_K1):
        g = g_ref[0, j]
        gp = g[:, 0:64]
        gx = g[:, 64:67]
        dx = gx - q3
        euc = jnp.sqrt(jnp.sum(dx * dx, axis=1, keepdims=True) + 1e-20)
        xyzfeat = jnp.concatenate([q3, gx, dx, euc], axis=1)
        y = (jnp.dot(xyzfeat, w1a_ref[...], preferred_element_type=F32)
             + jnp.dot(wp, w1b_ref[...], preferred_element_type=F32)
             + jnp.dot(gp, w1c_ref[...], preferred_element_type=F32))
        ye = jnp.dot(xyzfeat, we_ref[...], preferred_element_type=F32)
        y1_ref[0, j] = y
        ye_ref[0, j] = ye
        s1s = s1s + jnp.sum(y, axis=0, keepdims=True)
        s1q = s1q + jnp.sum(y * y, axis=0, keepdims=True)
        ses = ses + jnp.sum(ye, axis=0, keepdims=True)
        seq = seq + jnp.sum(ye * ye, axis=0, keepdims=True)
    s1_ref[0:1, :] = s1_ref[0:1, :] + s1s
    s1_ref[1:2, :] = s1_ref[1:2, :] + s1q
    se_ref[0:1, :] = se_ref[0:1, :] + ses
    se_ref[1:2, :] = se_ref[1:2, :] + seq


def _p1_call(xyz8, wpoints, g1, w1a, w1b, w1c, we):
    b, k, n, _ = g1.shape
    co, ce = w1a.shape[1], we.shape[1]
    return pl.pallas_call(
        _p1_body,
        grid=(b, n // _TN),
        in_specs=[
            pl.BlockSpec((1, _TN, 8), lambda bb, i: (bb, i, 0)),
            pl.BlockSpec((1, _TN, wpoints.shape[2]), lambda bb, i: (bb, i, 0)),
            pl.BlockSpec((1, k, _TN, _TW), lambda bb, i: (bb, 0, i, 0)),
            pl.BlockSpec(w1a.shape, lambda bb, i: (0, 0)),
            pl.BlockSpec(w1b.shape, lambda bb, i: (0, 0)),
            pl.BlockSpec(w1c.shape, lambda bb, i: (0, 0)),
            pl.BlockSpec(we.shape, lambda bb, i: (0, 0)),
        ],
        out_specs=[
            pl.BlockSpec((1, k, _TN, co), lambda bb, i: (bb, 0, i, 0)),
            pl.BlockSpec((1, k, _TN, ce), lambda bb, i: (bb, 0, i, 0)),
            pl.BlockSpec((8, co), lambda bb, i: (0, 0)),
            pl.BlockSpec((8, ce), lambda bb, i: (0, 0)),
        ],
        out_shape=[
            jax.ShapeDtypeStruct((b, k, n, co), F32),
            jax.ShapeDtypeStruct((b, k, n, ce), F32),
            jax.ShapeDtypeStruct((8, co), F32),
            jax.ShapeDtypeStruct((8, ce), F32),
        ],
    )(xyz8, wpoints, g1, w1a, w1b, w1c, we)


def _layer_body(yin_ref, p_ref, w_ref, yout_ref, s_ref, *, k):
    @pl.when(_first_step())
    def _():
        s_ref[...] = jnp.zeros_like(s_ref)

    sc = p_ref[0:1, :]
    sh = p_ref[1:2, :]
    ss = jnp.zeros((1, w_ref.shape[1]), F32)
    sq = jnp.zeros((1, w_ref.shape[1]), F32)
    for j in range(k):
        x = jnp.maximum(yin_ref[0, j] * sc + sh, 0.0)
        y = jnp.dot(x, w_ref[...], preferred_element_type=F32)
        yout_ref[0, j] = y
        ss = ss + jnp.sum(y, axis=0, keepdims=True)
        sq = sq + jnp.sum(y * y, axis=0, keepdims=True)
    s_ref[0:1, :] = s_ref[0:1, :] + ss
    s_ref[1:2, :] = s_ref[1:2, :] + sq


def _layer_call(yin, p, w):
    b, k, n, cin = yin.shape
    co = w.shape[1]
    return pl.pallas_call(
        functools.partial(_layer_body, k=k),
        grid=(b, n // _TN),
        in_specs=[
            pl.BlockSpec((1, k, _TN, cin), lambda bb, i: (bb, 0, i, 0)),
            pl.BlockSpec((8, cin), lambda bb, i: (0, 0)),
            pl.BlockSpec(w.shape, lambda bb, i: (0, 0)),
        ],
        out_specs=[
            pl.BlockSpec((1, k, _TN, co), lambda bb, i: (bb, 0, i, 0)),
            pl.BlockSpec((8, co), lambda bb, i: (0, 0)),
        ],
        out_shape=[
            jax.ShapeDtypeStruct((b, k, n, co), F32),
            jax.ShapeDtypeStruct((8, co), F32),
        ],
    )(yin, p, w)


def _p4_body(ye_ref, pe_ref, y3_ref, p3_ref, wa_ref, wb_ref,
             y4_ref, s4_ref, *, k):
    @pl.when(_first_step())
    def _():
        s4_ref[...] = jnp.zeros_like(s4_ref)

    esc, esh = pe_ref[0:1, :], pe_ref[1:2, :]
    xsc, xsh = p3_ref[0:1, :], p3_ref[1:2, :]
    ss = jnp.zeros((1, wa_ref.shape[1]), F32)
    sq = jnp.zeros((1, wa_ref.shape[1]), F32)
    for j in range(k):
        xe = jnp.maximum(ye_ref[0, j] * esc + esh, 0.0)
        x3 = jnp.maximum(y3_ref[0, j] * xsc + xsh, 0.0)
        y = (jnp.dot(xe, wa_ref[...], preferred_element_type=F32)
             + jnp.dot(x3, wb_ref[...], preferred_element_type=F32))
        y4_ref[0, j] = y
        ss = ss + jnp.sum(y, axis=0, keepdims=True)
        sq = sq + jnp.sum(y * y, axis=0, keepdims=True)
    s4_ref[0:1, :] = s4_ref[0:1, :] + ss
    s4_ref[1:2, :] = s4_ref[1:2, :] + sq


def _p4_call(ye, pe, y3, p3, wa, wb):
    b, k, n, ce = ye.shape
    co = wa.shape[1]
    return pl.pallas_call(
        functools.partial(_p4_body, k=k),
        grid=(b, n // _TN),
        in_specs=[
            pl.BlockSpec((1, k, _TN, ce), lambda bb, i: (bb, 0, i, 0)),
            pl.BlockSpec((8, ce), lambda bb, i: (0, 0)),
            pl.BlockSpec((1, k, _TN, y3.shape[3]), lambda bb, i: (bb, 0, i, 0)),
            pl.BlockSpec((8, y3.shape[3]), lambda bb, i: (0, 0)),
            pl.BlockSpec(wa.shape, lambda bb, i: (0, 0)),
            pl.BlockSpec(wb.shape, lambda bb, i: (0, 0)),
        ],
        out_specs=[
            pl.BlockSpec((1, k, _TN, co), lambda bb, i: (bb, 0, i, 0)),
            pl.BlockSpec((8, co), lambda bb, i: (0, 0)),
        ],
        out_shape=[
            jax.ShapeDtypeStruct((b, k, n, co), F32),
            jax.ShapeDtypeStruct((8, co), F32),
        ],
    )(ye, pe, y3, p3, wa, wb)


def _p6_body(y5_ref, p5_ref, y3_ref, p3_ref, out_ref, *, k):
    wsc, wsh = p5_ref[0:1, :], p5_ref[1:2, :]
    xsc, xsh = p3_ref[0:1, :], p3_ref[1:2, :]
    zs = [jnp.maximum(y5_ref[0, j] * wsc + wsh, 0.0) for j in range(k)]
    xs = [jnp.maximum(y3_ref[0, j] * xsc + xsh, 0.0) for j in range(k)]
    m = zs[0]
    for j in range(1, k):
        m = jnp.maximum(m, zs[j])
    es = [jnp.exp(z - m) for z in zs]
    tot = es[0]
    for j in range(1, k):
        tot = tot + es[j]
    acc = es[0] * xs[0]
    for j in range(1, k):
        acc = acc + es[j] * xs[j]
    out_ref[0] = acc / tot


def _p6_call(y5, p5, y3, p3):
    b, k, n, c = y5.shape
    return pl.pallas_call(
        functools.partial(_p6_body, k=k),
        grid=(b, n // _TN),
        in_specs=[
            pl.BlockSpec((1, k, _TN, c), lambda bb, i: (bb, 0, i, 0)),
            pl.BlockSpec((8, c), lambda bb, i: (0, 0)),
            pl.BlockSpec((1, k, _TN, y3.shape[3]), lambda bb, i: (bb, 0, i, 0)),
            pl.BlockSpec((8, y3.shape[3]), lambda bb, i: (0, 0)),
        ],
        out_specs=pl.BlockSpec((1, _TN, y3.shape[3]), lambda bb, i: (bb, i, 0)),
        out_shape=jax.ShapeDtypeStruct((b, n, y3.shape[3]), F32),
    )(y5, p5, y3, p3)


def _q1_body(xyz_ref, g_ref, we_ref, ye_ref, se_ref, *, k):
    @pl.when(_first_step())
    def _():
        se_ref[...] = jnp.zeros_like(se_ref)

    q3 = xyz_ref[0][:, 0:3]
    ss = jnp.zeros((1, we_ref.shape[1]), F32)
    sq = jnp.zeros((1, we_ref.shape[1]), F32)
    for j in range(k):
        g = g_ref[0, j]
        gx = g[:, 64:67]
        dx = gx - q3
        euc = jnp.sqrt(jnp.sum(dx * dx, axis=1, keepdims=True) + 1e-20)
        xyzfeat = jnp.concatenate([q3, gx, dx, euc], axis=1)
        y = jnp.dot(xyzfeat, we_ref[...], preferred_element_type=F32)
        ye_ref[0, j] = y
        ss = ss + jnp.sum(y, axis=0, keepdims=True)
        sq = sq + jnp.sum(y * y, axis=0, keepdims=True)
    se_ref[0:1, :] = se_ref[0:1, :] + ss
    se_ref[1:2, :] = se_ref[1:2, :] + sq


def _q1_call(xyz8, g2, we):
    b, k, n, _ = g2.shape
    co = we.shape[1]
    return pl.pallas_call(
        functools.partial(_q1_body, k=k),
        grid=(b, n // _TN),
        in_specs=[
            pl.BlockSpec((1, _TN, 8), lambda bb, i: (bb, i, 0)),
            pl.BlockSpec((1, k, _TN, _TW), lambda bb, i: (bb, 0, i, 0)),
            pl.BlockSpec(we.shape, lambda bb, i: (0, 0)),
        ],
        out_specs=[
            pl.BlockSpec((1, k, _TN, co), lambda bb, i: (bb, 0, i, 0)),
            pl.BlockSpec((8, co), lambda bb, i: (0, 0)),
        ],
        out_shape=[
            jax.ShapeDtypeStruct((b, k, n, co), F32),
            jax.ShapeDtypeStruct((8, co), F32),
        ],
    )(xyz8, g2, we)


def _q2_body(ye_ref, pe_ref, wp_ref, g_ref, wa_ref, wb_ref, wc_ref,
             y6_ref, s6_ref, *, k):
    @pl.when(_first_step())
    def _():
        s6_ref[...] = jnp.zeros_like(s6_ref)

    esc, esh = pe_ref[0:1, :], pe_ref[1:2, :]
    wp = wp_ref[0]
    wpy = jnp.dot(wp, wb_ref[...], preferred_element_type=F32)
    ss = jnp.zeros((1, wa_ref.shape[1]), F32)
    sq = jnp.zeros((1, wa_ref.shape[1]), F32)
    for j in range(k):
        xe = jnp.maximum(ye_ref[0, j] * esc + esh, 0.0)
        gp = g_ref[0, j][:, 0:64]
        y = (jnp.dot(xe, wa_ref[...], preferred_element_type=F32)
             + wpy
             + jnp.dot(gp, wc_ref[...], preferred_element_type=F32))
        y6_ref[0, j] = y
        ss = ss + jnp.sum(y, axis=0, keepdims=True)
        sq = sq + jnp.sum(y * y, axis=0, keepdims=True)
    s6_ref[0:1, :] = s6_ref[0:1, :] + ss
    s6_ref[1:2, :] = s6_ref[1:2, :] + sq


def _q2_call(ye2, pe2, wpoints, g2, wa, wb, wc):
    b, k, n, ce = ye2.shape
    co = wa.shape[1]
    return pl.pallas_call(
        functools.partial(_q2_body, k=k),
        grid=(b, n // _TN),
        in_specs=[
            pl.BlockSpec((1, k, _TN, ce), lambda bb, i: (bb, 0, i, 0)),
            pl.BlockSpec((8, ce), lambda bb, i: (0, 0)),
            pl.BlockSpec((1, _TN, wpoints.shape[2]), lambda bb, i: (bb, i, 0)),
            pl.BlockSpec((1, k, _TN, _TW), lambda bb, i: (bb, 0, i, 0)),
            pl.BlockSpec(wa.shape, lambda bb, i: (0, 0)),
            pl.BlockSpec(wb.shape, lambda bb, i: (0, 0)),
            pl.BlockSpec(wc.shape, lambda bb, i: (0, 0)),
        ],
        out_specs=[
            pl.BlockSpec((1, k, _TN, co), lambda bb, i: (bb, 0, i, 0)),
            pl.BlockSpec((8, co), lambda bb, i: (0, 0)),
        ],
        out_shape=[
            jax.ShapeDtypeStruct((b, k, n, co), F32),
            jax.ShapeDtypeStruct((8, co), F32),
        ],
    )(ye2, pe2, wpoints, g2, wa, wb, wc)


def _q4_body(y7_ref, p7_ref, g_ref, out_ref, *, k):
    wsc, wsh = p7_ref[0:1, :], p7_ref[1:2, :]
    zs = [jnp.maximum(y7_ref[0, j] * wsc + wsh, 0.0) for j in range(k)]
    xs = [g_ref[0, j][:, 0:64] for j in range(k)]
    m = zs[0]
    for j in range(1, k):
        m = jnp.maximum(m, zs[j])
    es = [jnp.exp(z - m) for z in zs]
    tot = es[0]
    for j in range(1, k):
        tot = tot + es[j]
    acc = es[0] * xs[0]
    for j in range(1, k):
        acc = acc + es[j] * xs[j]
    out_ref[0] = acc / tot


def _q4_call(y7, p7, g2):
    b, k, n, c = y7.shape
    return pl.pallas_call(
        functools.partial(_q4_body, k=k),
        grid=(b, n // _TN),
        in_specs=[
            pl.BlockSpec((1, k, _TN, c), lambda bb, i: (bb, 0, i, 0)),
            pl.BlockSpec((8, c), lambda bb, i: (0, 0)),
            pl.BlockSpec((1, k, _TN, _TW), lambda bb, i: (bb, 0, i, 0)),
        ],
        out_specs=pl.BlockSpec((1, _TN, c), lambda bb, i: (bb, i, 0)),
        out_shape=jax.ShapeDtypeStruct((b, n, c), F32),
    )(y7, p7, g2)


def _ss(stats, gb, cnt):
    gamma, beta = gb
    mean = stats[0] / cnt
    var = stats[1] / cnt - mean * mean
    sc = gamma / jnp.sqrt(var + 1e-5)
    sh = beta - mean * sc
    return jnp.concatenate(
        [sc[None], sh[None], jnp.zeros((6,) + sc.shape, F32)], axis=0)


def kernel(warped_xyz, f2_xyz, warped_points, f2_points, params):
    b, n, _ = warped_xyz.shape
    m = f2_xyz.shape[1]

    pad_n = jnp.zeros((b, n, 5), F32)
    pad_m = jnp.zeros((b, m, 5), F32)
    wxyz8 = jnp.concatenate([warped_xyz, pad_n], axis=-1)
    fxyz8 = jnp.concatenate([f2_xyz, pad_m], axis=-1)
    fxyzT = jnp.swapaxes(fxyz8, 1, 2)
    wxyzT = jnp.swapaxes(wxyz8, 1, 2)

    idx1 = _knn_call(wxyz8, fxyzT, _K1)
    idxq = jnp.swapaxes(idx1, 1, 2).reshape(-1)
    table1 = jnp.concatenate(
        [f2_points, fxyz8], axis=-1)
    table1 = jnp.concatenate(
        [table1, jnp.zeros((b, m, _TW - table1.shape[-1]), F32)],
        axis=-1).reshape(b * m, _TW)
    g1 = _gather_rows(table1, idxq).reshape(b, _K1, n, _TW)

    w1 = params['conv1'][0][0]
    y1, ye, s1, se = _p1_call(wxyz8, warped_points, g1,
                              w1[0:10], w1[10:74], w1[74:138],
                              params['enc1'][0])
    cnt1 = float(b * n * _K1)
    y2, s2 = _layer_call(y1, _ss(s1, params['bn1s'][0], cnt1),
                         params['conv1'][1][0])
    y3, s3 = _layer_call(y2, _ss(s2, params['bn1s'][1], cnt1),
                         params['conv1'][2][0])
    p3 = _ss(s3, params['bn1s'][2], cnt1)
    pe = _ss(se, params['bn_e1'], cnt1)
    w4 = params['conv2'][0][0]
    y4, s4 = _p4_call(ye, pe, y3, p3, w4[0:64], w4[64:128])
    y5, s5 = _layer_call(y4, _ss(s4, params['bn2s'][0], cnt1),
                         params['conv2'][1][0])
    p5 = _ss(s5, params['bn2s'][1], cnt1)
    feat1 = _p6_call(y5, p5, y3, p3)

    idx2 = _knn_call(wxyz8, wxyzT, _K2)
    idxp = jnp.swapaxes(idx2, 1, 2).reshape(-1)
    table2 = jnp.concatenate(
        [feat1, wxyz8, jnp.zeros((b, n, _TW - 72), F32)],
        axis=-1).reshape(b * n, _TW)
    g2 = _gather_rows(table2, idxp).reshape(b, _K2, n, _TW)

    ye2, se2 = _q1_call(wxyz8, g2, params['enc2'][0])
    cnt2 = float(b * n * _K2)
    pe2 = _ss(se2, params['bn_e2'], cnt2)
    w6 = params['conv3'][0][0]
    y6, s6 = _q2_call(ye2, pe2, warped_points, g2,
                      w6[0:64], w6[64:128], w6[128:192])
    y7, s7 = _layer_call(y6, _ss(s6, params['bn2s'][0], cnt2),
                         params['conv3'][1][0])
    p7 = _ss(s7, params['bn2s'][1], cnt2)
    return _q4_call(y7, p7, g2)


# speedup vs baseline: 22.2669x; 1.0689x over previous
import functools

import jax
import jax.numpy as jnp
from jax import lax
from jax.experimental import pallas as pl
from jax.experimental.pallas import tpu as pltpu
from jax.experimental.pallas import tpu_sc as plsc

F32 = jnp.float32
_TN = 256
_K1 = 6
_K2 = 4
_TW = 128
_NW = 32
_CHUNK = 128


def _first_step():
    return (pl.program_id(0) == 0) & (pl.program_id(1) == 0)


def _knn_body(q_ref, xt_ref, idx_ref, *, k, m):
    q = q_ref[0]
    xt = xt_ref[0]
    d = -2.0 * jnp.dot(q, xt, preferred_element_type=F32)
    d = d + jnp.sum(q * q, axis=1, keepdims=True)
    d = d + jnp.sum(xt * xt, axis=0, keepdims=True)
    iota = lax.broadcasted_iota(jnp.int32, d.shape, 1)
    cols = []
    for _ in range(k):
        am = jnp.argmin(d, axis=1)[:, None]
        cols.append(am)
        d = jnp.where(iota == am, jnp.inf, d)
    idx_ref[0] = jnp.concatenate(cols, axis=1) + pl.program_id(0) * m


def _knn_call(q8, xt8, k):
    b, n, _ = q8.shape
    m = xt8.shape[2]
    return pl.pallas_call(
        functools.partial(_knn_body, k=k, m=m),
        grid=(b, n // _TN),
        in_specs=[
            pl.BlockSpec((1, _TN, 8), lambda bb, i: (bb, i, 0)),
            pl.BlockSpec((1, 8, m), lambda bb, i: (bb, 0, 0)),
        ],
        out_specs=pl.BlockSpec((1, _TN, k), lambda bb, i: (bb, i, 0)),
        out_shape=jax.ShapeDtypeStruct((b, n, k), jnp.int32),
    )(q8, xt8)


def _gather_rows(table, idx):
    rows = idx.shape[0]
    per_w = rows // _NW
    chunks = per_w // _CHUNK
    idx3 = idx.reshape(_NW, chunks, _CHUNK)
    mesh = plsc.VectorSubcoreMesh(core_axis_name="c", subcore_axis_name="s",
                                  num_cores=2)

    @functools.partial(
        pl.kernel, mesh=mesh,
        out_type=jax.ShapeDtypeStruct((rows, _TW), F32),
        scratch_types=[
            pltpu.VMEM((chunks, _CHUNK), jnp.int32),
            pltpu.VMEM((2, _CHUNK, _TW), F32),
            pltpu.SemaphoreType.DMA,
            pltpu.SemaphoreType.DMA,
        ],
    )
    def gk(table_hbm, idx_hbm, out_hbm, idx_v, rows_v, sem0, sem1):
        wid = lax.axis_index("s") * 2 + lax.axis_index("c")
        base = wid * per_w
        pltpu.sync_copy(idx_hbm.at[wid], idx_v)
        sems = (sem0, sem1)
        prev = None
        for j in range(chunks):
            cp = pltpu.async_copy(table_hbm.at[idx_v.at[j]],
                                  rows_v.at[j % 2], sems[j % 2])
            if prev is not None:
                pj, pcp = prev
                pcp.wait()
                pltpu.sync_copy(
                    rows_v.at[pj % 2],
                    out_hbm.at[pl.ds(base + pj * _CHUNK, _CHUNK)])
            prev = (j, cp)
        pj, pcp = prev
        pcp.wait()
        pltpu.sync_copy(rows_v.at[pj % 2],
                        out_hbm.at[pl.ds(base + pj * _CHUNK, _CHUNK)])

    return gk(table, idx3)


def _p1_body(xyz_ref, wp_ref, g_ref, w1a_ref, w1b_ref, w1c_ref, we_ref,
             y1_ref, ye_ref, s1_ref, se_ref):
    @pl.when(_first_step())
    def _():
        s1_ref[...] = jnp.zeros_like(s1_ref)
        se_ref[...] = jnp.zeros_like(se_ref)

    q3 = xyz_ref[0][:, 0:3]
    wp = wp_ref[0]
    s1s = jnp.zeros((1, y1_ref.shape[-1]), F32)
    s1q = jnp.zeros((1, y1_ref.shape[-1]), F32)
    ses = jnp.zeros((1, ye_ref.shape[-1]), F32)
    seq = jnp.zeros((1, ye_ref.shape[-1]), F32)
    for j in range(_K1):
        g = g_ref[0, j]
        gp = g[:, 0:64]
        gx = g[:, 64:67]
        dx = gx - q3
        euc = jnp.sqrt(jnp.sum(dx * dx, axis=1, keepdims=True) + 1e-20)
        xyzfeat = jnp.concatenate([q3, gx, dx, euc], axis=1)
        y = (jnp.dot(xyzfeat, w1a_ref[...], preferred_element_type=F32)
             + jnp.dot(wp, w1b_ref[...], preferred_element_type=F32)
             + jnp.dot(gp, w1c_ref[...], preferred_element_type=F32))
        ye = jnp.dot(xyzfeat, we_ref[...], preferred_element_type=F32)
        y1_ref[0, j] = y
        ye_ref[0, j] = ye
        s1s = s1s + jnp.sum(y, axis=0, keepdims=True)
        s1q = s1q + jnp.sum(y * y, axis=0, keepdims=True)
        ses = ses + jnp.sum(ye, axis=0, keepdims=True)
        seq = seq + jnp.sum(ye * ye, axis=0, keepdims=True)
    s1_ref[0:1, :] = s1_ref[0:1, :] + s1s
    s1_ref[1:2, :] = s1_ref[1:2, :] + s1q
    se_ref[0:1, :] = se_ref[0:1, :] + ses
    se_ref[1:2, :] = se_ref[1:2, :] + seq


def _p1_call(xyz8, wpoints, g1, w1a, w1b, w1c, we):
    b, k, n, _ = g1.shape
    co, ce = w1a.shape[1], we.shape[1]
    return pl.pallas_call(
        _p1_body,
        grid=(b, n // _TN),
        in_specs=[
            pl.BlockSpec((1, _TN, 8), lambda bb, i: (bb, i, 0)),
            pl.BlockSpec((1, _TN, wpoints.shape[2]), lambda bb, i: (bb, i, 0)),
            pl.BlockSpec((1, k, _TN, _TW), lambda bb, i: (bb, 0, i, 0)),
            pl.BlockSpec(w1a.shape, lambda bb, i: (0, 0)),
            pl.BlockSpec(w1b.shape, lambda bb, i: (0, 0)),
            pl.BlockSpec(w1c.shape, lambda bb, i: (0, 0)),
            pl.BlockSpec(we.shape, lambda bb, i: (0, 0)),
        ],
        out_specs=[
            pl.BlockSpec((1, k, _TN, co), lambda bb, i: (bb, 0, i, 0)),
            pl.BlockSpec((1, k, _TN, ce), lambda bb, i: (bb, 0, i, 0)),
            pl.BlockSpec((8, co), lambda bb, i: (0, 0)),
            pl.BlockSpec((8, ce), lambda bb, i: (0, 0)),
        ],
        out_shape=[
            jax.ShapeDtypeStruct((b, k, n, co), F32),
            jax.ShapeDtypeStruct((b, k, n, ce), F32),
            jax.ShapeDtypeStruct((8, co), F32),
            jax.ShapeDtypeStruct((8, ce), F32),
        ],
    )(xyz8, wpoints, g1, w1a, w1b, w1c, we)


def _layer_body(yin_ref, p_ref, w_ref, yout_ref, s_ref, *, k):
    @pl.when(_first_step())
    def _():
        s_ref[...] = jnp.zeros_like(s_ref)

    sc = p_ref[0:1, :]
    sh = p_ref[1:2, :]
    ss = jnp.zeros((1, w_ref.shape[1]), F32)
    sq = jnp.zeros((1, w_ref.shape[1]), F32)
    for j in range(k):
        x = jnp.maximum(yin_ref[0, j] * sc + sh, 0.0)
        y = jnp.dot(x, w_ref[...], preferred_element_type=F32)
        yout_ref[0, j] = y
        ss = ss + jnp.sum(y, axis=0, keepdims=True)
        sq = sq + jnp.sum(y * y, axis=0, keepdims=True)
    s_ref[0:1, :] = s_ref[0:1, :] + ss
    s_ref[1:2, :] = s_ref[1:2, :] + sq


def _layer_call(yin, p, w):
    b, k, n, cin = yin.shape
    co = w.shape[1]
    return pl.pallas_call(
        functools.partial(_layer_body, k=k),
        grid=(b, n // _TN),
        in_specs=[
            pl.BlockSpec((1, k, _TN, cin), lambda bb, i: (bb, 0, i, 0)),
            pl.BlockSpec((8, cin), lambda bb, i: (0, 0)),
            pl.BlockSpec(w.shape, lambda bb, i: (0, 0)),
        ],
        out_specs=[
            pl.BlockSpec((1, k, _TN, co), lambda bb, i: (bb, 0, i, 0)),
            pl.BlockSpec((8, co), lambda bb, i: (0, 0)),
        ],
        out_shape=[
            jax.ShapeDtypeStruct((b, k, n, co), F32),
            jax.ShapeDtypeStruct((8, co), F32),
        ],
    )(yin, p, w)


def _p4_body(ye_ref, pe_ref, y3_ref, p3_ref, wa_ref, wb_ref,
             y4_ref, s4_ref, *, k):
    @pl.when(_first_step())
    def _():
        s4_ref[...] = jnp.zeros_like(s4_ref)

    esc, esh = pe_ref[0:1, :], pe_ref[1:2, :]
    xsc, xsh = p3_ref[0:1, :], p3_ref[1:2, :]
    ss = jnp.zeros((1, wa_ref.shape[1]), F32)
    sq = jnp.zeros((1, wa_ref.shape[1]), F32)
    for j in range(k):
        xe = jnp.maximum(ye_ref[0, j] * esc + esh, 0.0)
        x3 = jnp.maximum(y3_ref[0, j] * xsc + xsh, 0.0)
        y = (jnp.dot(xe, wa_ref[...], preferred_element_type=F32)
             + jnp.dot(x3, wb_ref[...], preferred_element_type=F32))
        y4_ref[0, j] = y
        ss = ss + jnp.sum(y, axis=0, keepdims=True)
        sq = sq + jnp.sum(y * y, axis=0, keepdims=True)
    s4_ref[0:1, :] = s4_ref[0:1, :] + ss
    s4_ref[1:2, :] = s4_ref[1:2, :] + sq


def _p4_call(ye, pe, y3, p3, wa, wb):
    b, k, n, ce = ye.shape
    co = wa.shape[1]
    return pl.pallas_call(
        functools.partial(_p4_body, k=k),
        grid=(b, n // _TN),
        in_specs=[
            pl.BlockSpec((1, k, _TN, ce), lambda bb, i: (bb, 0, i, 0)),
            pl.BlockSpec((8, ce), lambda bb, i: (0, 0)),
            pl.BlockSpec((1, k, _TN, y3.shape[3]), lambda bb, i: (bb, 0, i, 0)),
            pl.BlockSpec((8, y3.shape[3]), lambda bb, i: (0, 0)),
            pl.BlockSpec(wa.shape, lambda bb, i: (0, 0)),
            pl.BlockSpec(wb.shape, lambda bb, i: (0, 0)),
        ],
        out_specs=[
            pl.BlockSpec((1, k, _TN, co), lambda bb, i: (bb, 0, i, 0)),
            pl.BlockSpec((8, co), lambda bb, i: (0, 0)),
        ],
        out_shape=[
            jax.ShapeDtypeStruct((b, k, n, co), F32),
            jax.ShapeDtypeStruct((8, co), F32),
        ],
    )(ye, pe, y3, p3, wa, wb)


def _p6_body(y5_ref, p5_ref, y3_ref, p3_ref, out_ref, *, k):
    wsc, wsh = p5_ref[0:1, :], p5_ref[1:2, :]
    xsc, xsh = p3_ref[0:1, :], p3_ref[1:2, :]
    zs = [jnp.maximum(y5_ref[0, j] * wsc + wsh, 0.0) for j in range(k)]
    xs = [jnp.maximum(y3_ref[0, j] * xsc + xsh, 0.0) for j in range(k)]
    m = zs[0]
    for j in range(1, k):
        m = jnp.maximum(m, zs[j])
    es = [jnp.exp(z - m) for z in zs]
    tot = es[0]
    for j in range(1, k):
        tot = tot + es[j]
    acc = es[0] * xs[0]
    for j in range(1, k):
        acc = acc + es[j] * xs[j]
    out_ref[0] = acc / tot


def _p6_call(y5, p5, y3, p3):
    b, k, n, c = y5.shape
    return pl.pallas_call(
        functools.partial(_p6_body, k=k),
        grid=(b, n // _TN),
        in_specs=[
            pl.BlockSpec((1, k, _TN, c), lambda bb, i: (bb, 0, i, 0)),
            pl.BlockSpec((8, c), lambda bb, i: (0, 0)),
            pl.BlockSpec((1, k, _TN, y3.shape[3]), lambda bb, i: (bb, 0, i, 0)),
            pl.BlockSpec((8, y3.shape[3]), lambda bb, i: (0, 0)),
        ],
        out_specs=pl.BlockSpec((1, _TN, y3.shape[3]), lambda bb, i: (bb, i, 0)),
        out_shape=jax.ShapeDtypeStruct((b, n, y3.shape[3]), F32),
    )(y5, p5, y3, p3)


def _q1_body(xyz_ref, g_ref, we_ref, ye_ref, se_ref, *, k):
    @pl.when(_first_step())
    def _():
        se_ref[...] = jnp.zeros_like(se_ref)

    q3 = xyz_ref[0][:, 0:3]
    ss = jnp.zeros((1, we_ref.shape[1]), F32)
    sq = jnp.zeros((1, we_ref.shape[1]), F32)
    for j in range(k):
        g = g_ref[0, j]
        gx = g[:, 64:67]
        dx = gx - q3
        euc = jnp.sqrt(jnp.sum(dx * dx, axis=1, keepdims=True) + 1e-20)
        xyzfeat = jnp.concatenate([q3, gx, dx, euc], axis=1)
        y = jnp.dot(xyzfeat, we_ref[...], preferred_element_type=F32)
        ye_ref[0, j] = y
        ss = ss + jnp.sum(y, axis=0, keepdims=True)
        sq = sq + jnp.sum(y * y, axis=0, keepdims=True)
    se_ref[0:1, :] = se_ref[0:1, :] + ss
    se_ref[1:2, :] = se_ref[1:2, :] + sq


def _q1_call(xyz8, g2, we):
    b, k, n, _ = g2.shape
    co = we.shape[1]
    return pl.pallas_call(
        functools.partial(_q1_body, k=k),
        grid=(b, n // _TN),
        in_specs=[
            pl.BlockSpec((1, _TN, 8), lambda bb, i: (bb, i, 0)),
            pl.BlockSpec((1, k, _TN, _TW), lambda bb, i: (bb, 0, i, 0)),
            pl.BlockSpec(we.shape, lambda bb, i: (0, 0)),
        ],
        out_specs=[
            pl.BlockSpec((1, k, _TN, co), lambda bb, i: (bb, 0, i, 0)),
            pl.BlockSpec((8, co), lambda bb, i: (0, 0)),
        ],
        out_shape=[
            jax.ShapeDtypeStruct((b, k, n, co), F32),
            jax.ShapeDtypeStruct((8, co), F32),
        ],
    )(xyz8, g2, we)


def _q2_body(ye_ref, pe_ref, wp_ref, g_ref, wa_ref, wb_ref, wc_ref,
             y6_ref, s6_ref, *, k):
    @pl.when(_first_step())
    def _():
        s6_ref[...] = jnp.zeros_like(s6_ref)

    esc, esh = pe_ref[0:1, :], pe_ref[1:2, :]
    wp = wp_ref[0]
    wpy = jnp.dot(wp, wb_ref[...], preferred_element_type=F32)
    ss = jnp.zeros((1, wa_ref.shape[1]), F32)
    sq = jnp.zeros((1, wa_ref.shape[1]), F32)
    for j in range(k):
        xe = jnp.maximum(ye_ref[0, j] * esc + esh, 0.0)
        gp = g_ref[0, j][:, 0:64]
        y = (jnp.dot(xe, wa_ref[...], preferred_element_type=F32)
             + wpy
             + jnp.dot(gp, wc_ref[...], preferred_element_type=F32))
        y6_ref[0, j] = y
        ss = ss + jnp.sum(y, axis=0, keepdims=True)
        sq = sq + jnp.sum(y * y, axis=0, keepdims=True)
    s6_ref[0:1, :] = s6_ref[0:1, :] + ss
    s6_ref[1:2, :] = s6_ref[1:2, :] + sq


def _q2_call(ye2, pe2, wpoints, g2, wa, wb, wc):
    b, k, n, ce = ye2.shape
    co = wa.shape[1]
    return pl.pallas_call(
        functools.partial(_q2_body, k=k),
        grid=(b, n // _TN),
        in_specs=[
            pl.BlockSpec((1, k, _TN, ce), lambda bb, i: (bb, 0, i, 0)),
            pl.BlockSpec((8, ce), lambda bb, i: (0, 0)),
            pl.BlockSpec((1, _TN, wpoints.shape[2]), lambda bb, i: (bb, i, 0)),
            pl.BlockSpec((1, k, _TN, _TW), lambda bb, i: (bb, 0, i, 0)),
            pl.BlockSpec(wa.shape, lambda bb, i: (0, 0)),
            pl.BlockSpec(wb.shape, lambda bb, i: (0, 0)),
            pl.BlockSpec(wc.shape, lambda bb, i: (0, 0)),
        ],
        out_specs=[
            pl.BlockSpec((1, k, _TN, co), lambda bb, i: (bb, 0, i, 0)),
            pl.BlockSpec((8, co), lambda bb, i: (0, 0)),
        ],
        out_shape=[
            jax.ShapeDtypeStruct((b, k, n, co), F32),
            jax.ShapeDtypeStruct((8, co), F32),
        ],
    )(ye2, pe2, wpoints, g2, wa, wb, wc)


def _q4_body(y7_ref, p7_ref, g_ref, out_ref, *, k):
    wsc, wsh = p7_ref[0:1, :], p7_ref[1:2, :]
    zs = [jnp.maximum(y7_ref[0, j] * wsc + wsh, 0.0) for j in range(k)]
    xs = [g_ref[0, j][:, 0:64] for j in range(k)]
    m = zs[0]
    for j in range(1, k):
        m = jnp.maximum(m, zs[j])
    es = [jnp.exp(z - m) for z in zs]
    tot = es[0]
    for j in range(1, k):
        tot = tot + es[j]
    acc = es[0] * xs[0]
    for j in range(1, k):
        acc = acc + es[j] * xs[j]
    out_ref[0] = acc / tot


def _q4_call(y7, p7, g2):
    b, k, n, c = y7.shape
    return pl.pallas_call(
        functools.partial(_q4_body, k=k),
        grid=(b, n // _TN),
        in_specs=[
            pl.BlockSpec((1, k, _TN, c), lambda bb, i: (bb, 0, i, 0)),
            pl.BlockSpec((8, c), lambda bb, i: (0, 0)),
            pl.BlockSpec((1, k, _TN, _TW), lambda bb, i: (bb, 0, i, 0)),
        ],
        out_specs=pl.BlockSpec((1, _TN, c), lambda bb, i: (bb, i, 0)),
        out_shape=jax.ShapeDtypeStruct((b, n, c), F32),
    )(y7, p7, g2)


def _ss(stats, gb, cnt):
    gamma, beta = gb
    mean = stats[0] / cnt
    var = stats[1] / cnt - mean * mean
    sc = gamma / jnp.sqrt(var + 1e-5)
    sh = beta - mean * sc
    return jnp.concatenate(
        [sc[None], sh[None], jnp.zeros((6,) + sc.shape, F32)], axis=0)


def kernel(warped_xyz, f2_xyz, warped_points, f2_points, params):
    b, n, _ = warped_xyz.shape
    m = f2_xyz.shape[1]

    pad_n = jnp.zeros((b, n, 5), F32)
    pad_m = jnp.zeros((b, m, 5), F32)
    wxyz8 = jnp.concatenate([warped_xyz, pad_n], axis=-1)
    fxyz8 = jnp.concatenate([f2_xyz, pad_m], axis=-1)
    fxyzT = jnp.swapaxes(fxyz8, 1, 2)
    wxyzT = jnp.swapaxes(wxyz8, 1, 2)

    idx1 = _knn_call(wxyz8, fxyzT, _K1)
    idxq = jnp.swapaxes(idx1, 1, 2).reshape(-1)
    table1 = jnp.concatenate(
        [f2_points, fxyz8], axis=-1)
    table1 = jnp.concatenate(
        [table1, jnp.zeros((b, m, _TW - table1.shape[-1]), F32)],
        axis=-1).reshape(b * m, _TW)
    g1 = _gather_rows(table1, idxq).reshape(b, _K1, n, _TW)

    w1 = params['conv1'][0][0]
    y1, ye, s1, se = _p1_call(wxyz8, warped_points, g1,
                              w1[0:10], w1[10:74], w1[74:138],
                              params['enc1'][0])
    cnt1 = float(b * n * _K1)
    y2, s2 = _layer_call(y1, _ss(s1, params['bn1s'][0], cnt1),
                         params['conv1'][1][0])
    y3, s3 = _layer_call(y2, _ss(s2, params['bn1s'][1], cnt1),
                         params['conv1'][2][0])
    p3 = _ss(s3, params['bn1s'][2], cnt1)
    pe = _ss(se, params['bn_e1'], cnt1)
    w4 = params['conv2'][0][0]
    y4, s4 = _p4_call(ye, pe, y3, p3, w4[0:64], w4[64:128])
    y5, s5 = _layer_call(y4, _ss(s4, params['bn2s'][0], cnt1),
                         params['conv2'][1][0])
    p5 = _ss(s5, params['bn2s'][1], cnt1)
    feat1 = _p6_call(y5, p5, y3, p3)

    idx2 = _knn_call(wxyz8, wxyzT, _K2)
    idxp = jnp.swapaxes(idx2, 1, 2).reshape(-1)
    table2 = jnp.concatenate(
        [feat1, wxyz8, jnp.zeros((b, n, _TW - 72), F32)],
        axis=-1).reshape(b * n, _TW)
    g2 = _gather_rows(table2, idxp).reshape(b, _K2, n, _TW)

    ye2, se2 = _q1_call(wxyz8, g2, params['enc2'][0])
    cnt2 = float(b * n * _K2)
    pe2 = _ss(se2, params['bn_e2'], cnt2)
    w6 = params['conv3'][0][0]
    y6, s6 = _q2_call(ye2, pe2, warped_points, g2,
                      w6[0:64], w6[64:128], w6[128:192])
    y7, s7 = _layer_call(y6, _ss(s6, params['bn2s'][0], cnt2),
                         params['conv3'][1][0])
    p7 = _ss(s7, params['bn2s'][1], cnt2)
    return _q4_call(y7, p7, g2)


# speedup vs baseline: 25.7096x; 1.1546x over previous
import functools

import jax
import jax.numpy as jnp
from jax import lax
from jax.experimental import pallas as pl
from jax.experimental.pallas import tpu as pltpu
from jax.experimental.pallas import tpu_sc as plsc

F32 = jnp.float32
_TN = 512
_K1 = 6
_K2 = 4
_TW = 128
_NW = 32
_CHUNK = 128


def _first_step():
    return (pl.program_id(0) == 0) & (pl.program_id(1) == 0)


def _knn_body(q_ref, xt_ref, idx_ref, *, k, m):
    q = q_ref[0]
    xt = xt_ref[0]
    d = -2.0 * jnp.dot(q, xt, preferred_element_type=F32)
    d = d + jnp.sum(q * q, axis=1, keepdims=True)
    d = d + jnp.sum(xt * xt, axis=0, keepdims=True)
    iota = lax.broadcasted_iota(jnp.int32, d.shape, 1)
    cols = []
    for _ in range(k):
        am = jnp.argmin(d, axis=1)[:, None]
        cols.append(am)
        d = jnp.where(iota == am, jnp.inf, d)
    idx_ref[0] = jnp.concatenate(cols, axis=1) + pl.program_id(0) * m


def _knn_call(q8, xt8, k):
    b, n, _ = q8.shape
    m = xt8.shape[2]
    return pl.pallas_call(
        functools.partial(_knn_body, k=k, m=m),
        grid=(b, n // _TN),
        in_specs=[
            pl.BlockSpec((1, _TN, 8), lambda bb, i: (bb, i, 0)),
            pl.BlockSpec((1, 8, m), lambda bb, i: (bb, 0, 0)),
        ],
        out_specs=pl.BlockSpec((1, _TN, k), lambda bb, i: (bb, i, 0)),
        out_shape=jax.ShapeDtypeStruct((b, n, k), jnp.int32),
    )(q8, xt8)


def _gather_rows(table, idx):
    rows = idx.shape[0]
    per_w = rows // _NW
    chunks = per_w // _CHUNK
    idx3 = idx.reshape(_NW, chunks, _CHUNK)
    mesh = plsc.VectorSubcoreMesh(core_axis_name="c", subcore_axis_name="s",
                                  num_cores=2)

    @functools.partial(
        pl.kernel, mesh=mesh,
        out_type=jax.ShapeDtypeStruct((rows, _TW), F32),
        scratch_types=[
            pltpu.VMEM((chunks, _CHUNK), jnp.int32),
            pltpu.VMEM((2, _CHUNK, _TW), F32),
            pltpu.SemaphoreType.DMA,
            pltpu.SemaphoreType.DMA,
        ],
    )
    def gk(table_hbm, idx_hbm, out_hbm, idx_v, rows_v, sem0, sem1):
        wid = lax.axis_index("s") * 2 + lax.axis_index("c")
        base = wid * per_w
        pltpu.sync_copy(idx_hbm.at[wid], idx_v)
        sems = (sem0, sem1)
        prev = None
        for j in range(chunks):
            cp = pltpu.async_copy(table_hbm.at[idx_v.at[j]],
                                  rows_v.at[j % 2], sems[j % 2])
            if prev is not None:
                pj, pcp = prev
                pcp.wait()
                pltpu.sync_copy(
                    rows_v.at[pj % 2],
                    out_hbm.at[pl.ds(base + pj * _CHUNK, _CHUNK)])
            prev = (j, cp)
        pj, pcp = prev
        pcp.wait()
        pltpu.sync_copy(rows_v.at[pj % 2],
                        out_hbm.at[pl.ds(base + pj * _CHUNK, _CHUNK)])

    return gk(table, idx3)


def _p1_body(xyz_ref, wp_ref, g_ref, w1a_ref, w1b_ref, w1c_ref, we_ref,
             y1_ref, ye_ref, s1_ref, se_ref):
    @pl.when(_first_step())
    def _():
        s1_ref[...] = jnp.zeros_like(s1_ref)
        se_ref[...] = jnp.zeros_like(se_ref)

    q3 = xyz_ref[0][:, 0:3]
    wp = wp_ref[0]
    s1s = jnp.zeros((1, y1_ref.shape[-1]), F32)
    s1q = jnp.zeros((1, y1_ref.shape[-1]), F32)
    ses = jnp.zeros((1, ye_ref.shape[-1]), F32)
    seq = jnp.zeros((1, ye_ref.shape[-1]), F32)
    for j in range(_K1):
        g = g_ref[0, j]
        gp = g[:, 0:64]
        gx = g[:, 64:67]
        dx = gx - q3
        euc = jnp.sqrt(jnp.sum(dx * dx, axis=1, keepdims=True) + 1e-20)
        xyzfeat = jnp.concatenate([q3, gx, dx, euc], axis=1)
        y = (jnp.dot(xyzfeat, w1a_ref[...], preferred_element_type=F32)
             + jnp.dot(wp, w1b_ref[...], preferred_element_type=F32)
             + jnp.dot(gp, w1c_ref[...], preferred_element_type=F32))
        ye = jnp.dot(xyzfeat, we_ref[...], preferred_element_type=F32)
        y1_ref[0, j] = y
        ye_ref[0, j] = ye
        s1s = s1s + jnp.sum(y, axis=0, keepdims=True)
        s1q = s1q + jnp.sum(y * y, axis=0, keepdims=True)
        ses = ses + jnp.sum(ye, axis=0, keepdims=True)
        seq = seq + jnp.sum(ye * ye, axis=0, keepdims=True)
    s1_ref[0:1, :] = s1_ref[0:1, :] + s1s
    s1_ref[1:2, :] = s1_ref[1:2, :] + s1q
    se_ref[0:1, :] = se_ref[0:1, :] + ses
    se_ref[1:2, :] = se_ref[1:2, :] + seq


def _p1_call(xyz8, wpoints, g1, w1a, w1b, w1c, we):
    b, k, n, _ = g1.shape
    co, ce = w1a.shape[1], we.shape[1]
    return pl.pallas_call(
        _p1_body,
        grid=(b, n // _TN),
        in_specs=[
            pl.BlockSpec((1, _TN, 8), lambda bb, i: (bb, i, 0)),
            pl.BlockSpec((1, _TN, wpoints.shape[2]), lambda bb, i: (bb, i, 0)),
            pl.BlockSpec((1, k, _TN, _TW), lambda bb, i: (bb, 0, i, 0)),
            pl.BlockSpec(w1a.shape, lambda bb, i: (0, 0)),
            pl.BlockSpec(w1b.shape, lambda bb, i: (0, 0)),
            pl.BlockSpec(w1c.shape, lambda bb, i: (0, 0)),
            pl.BlockSpec(we.shape, lambda bb, i: (0, 0)),
        ],
        out_specs=[
            pl.BlockSpec((1, k, _TN, co), lambda bb, i: (bb, 0, i, 0)),
            pl.BlockSpec((1, k, _TN, ce), lambda bb, i: (bb, 0, i, 0)),
            pl.BlockSpec((8, co), lambda bb, i: (0, 0)),
            pl.BlockSpec((8, ce), lambda bb, i: (0, 0)),
        ],
        out_shape=[
            jax.ShapeDtypeStruct((b, k, n, co), F32),
            jax.ShapeDtypeStruct((b, k, n, ce), F32),
            jax.ShapeDtypeStruct((8, co), F32),
            jax.ShapeDtypeStruct((8, ce), F32),
        ],
    )(xyz8, wpoints, g1, w1a, w1b, w1c, we)


def _layer_body(yin_ref, p_ref, w_ref, yout_ref, s_ref, *, k):
    @pl.when(_first_step())
    def _():
        s_ref[...] = jnp.zeros_like(s_ref)

    sc = p_ref[0:1, :]
    sh = p_ref[1:2, :]
    ss = jnp.zeros((1, w_ref.shape[1]), F32)
    sq = jnp.zeros((1, w_ref.shape[1]), F32)
    for j in range(k):
        x = jnp.maximum(yin_ref[0, j] * sc + sh, 0.0)
        y = jnp.dot(x, w_ref[...], preferred_element_type=F32)
        yout_ref[0, j] = y
        ss = ss + jnp.sum(y, axis=0, keepdims=True)
        sq = sq + jnp.sum(y * y, axis=0, keepdims=True)
    s_ref[0:1, :] = s_ref[0:1, :] + ss
    s_ref[1:2, :] = s_ref[1:2, :] + sq


def _layer_call(yin, p, w):
    b, k, n, cin = yin.shape
    co = w.shape[1]
    return pl.pallas_call(
        functools.partial(_layer_body, k=k),
        grid=(b, n // _TN),
        in_specs=[
            pl.BlockSpec((1, k, _TN, cin), lambda bb, i: (bb, 0, i, 0)),
            pl.BlockSpec((8, cin), lambda bb, i: (0, 0)),
            pl.BlockSpec(w.shape, lambda bb, i: (0, 0)),
        ],
        out_specs=[
            pl.BlockSpec((1, k, _TN, co), lambda bb, i: (bb, 0, i, 0)),
            pl.BlockSpec((8, co), lambda bb, i: (0, 0)),
        ],
        out_shape=[
            jax.ShapeDtypeStruct((b, k, n, co), F32),
            jax.ShapeDtypeStruct((8, co), F32),
        ],
    )(yin, p, w)


def _p4_body(ye_ref, pe_ref, y3_ref, p3_ref, wa_ref, wb_ref,
             y4_ref, s4_ref, *, k):
    @pl.when(_first_step())
    def _():
        s4_ref[...] = jnp.zeros_like(s4_ref)

    esc, esh = pe_ref[0:1, :], pe_ref[1:2, :]
    xsc, xsh = p3_ref[0:1, :], p3_ref[1:2, :]
    ss = jnp.zeros((1, wa_ref.shape[1]), F32)
    sq = jnp.zeros((1, wa_ref.shape[1]), F32)
    for j in range(k):
        xe = jnp.maximum(ye_ref[0, j] * esc + esh, 0.0)
        x3 = jnp.maximum(y3_ref[0, j] * xsc + xsh, 0.0)
        y = (jnp.dot(xe, wa_ref[...], preferred_element_type=F32)
             + jnp.dot(x3, wb_ref[...], preferred_element_type=F32))
        y4_ref[0, j] = y
        ss = ss + jnp.sum(y, axis=0, keepdims=True)
        sq = sq + jnp.sum(y * y, axis=0, keepdims=True)
    s4_ref[0:1, :] = s4_ref[0:1, :] + ss
    s4_ref[1:2, :] = s4_ref[1:2, :] + sq


def _p4_call(ye, pe, y3, p3, wa, wb):
    b, k, n, ce = ye.shape
    co = wa.shape[1]
    return pl.pallas_call(
        functools.partial(_p4_body, k=k),
        grid=(b, n // _TN),
        in_specs=[
            pl.BlockSpec((1, k, _TN, ce), lambda bb, i: (bb, 0, i, 0)),
            pl.BlockSpec((8, ce), lambda bb, i: (0, 0)),
            pl.BlockSpec((1, k, _TN, y3.shape[3]), lambda bb, i: (bb, 0, i, 0)),
            pl.BlockSpec((8, y3.shape[3]), lambda bb, i: (0, 0)),
            pl.BlockSpec(wa.shape, lambda bb, i: (0, 0)),
            pl.BlockSpec(wb.shape, lambda bb, i: (0, 0)),
        ],
        out_specs=[
            pl.BlockSpec((1, k, _TN, co), lambda bb, i: (bb, 0, i, 0)),
            pl.BlockSpec((8, co), lambda bb, i: (0, 0)),
        ],
        out_shape=[
            jax.ShapeDtypeStruct((b, k, n, co), F32),
            jax.ShapeDtypeStruct((8, co), F32),
        ],
    )(ye, pe, y3, p3, wa, wb)


def _p6_body(y5_ref, p5_ref, y3_ref, p3_ref, out_ref, *, k):
    wsc, wsh = p5_ref[0:1, :], p5_ref[1:2, :]
    xsc, xsh = p3_ref[0:1, :], p3_ref[1:2, :]
    zs = [jnp.maximum(y5_ref[0, j] * wsc + wsh, 0.0) for j in range(k)]
    xs = [jnp.maximum(y3_ref[0, j] * xsc + xsh, 0.0) for j in range(k)]
    m = zs[0]
    for j in range(1, k):
        m = jnp.maximum(m, zs[j])
    es = [jnp.exp(z - m) for z in zs]
    tot = es[0]
    for j in range(1, k):
        tot = tot + es[j]
    acc = es[0] * xs[0]
    for j in range(1, k):
        acc = acc + es[j] * xs[j]
    out_ref[0] = acc / tot


def _p6_call(y5, p5, y3, p3):
    b, k, n, c = y5.shape
    return pl.pallas_call(
        functools.partial(_p6_body, k=k),
        grid=(b, n // _TN),
        in_specs=[
            pl.BlockSpec((1, k, _TN, c), lambda bb, i: (bb, 0, i, 0)),
            pl.BlockSpec((8, c), lambda bb, i: (0, 0)),
            pl.BlockSpec((1, k, _TN, y3.shape[3]), lambda bb, i: (bb, 0, i, 0)),
            pl.BlockSpec((8, y3.shape[3]), lambda bb, i: (0, 0)),
        ],
        out_specs=pl.BlockSpec((1, _TN, y3.shape[3]), lambda bb, i: (bb, i, 0)),
        out_shape=jax.ShapeDtypeStruct((b, n, y3.shape[3]), F32),
    )(y5, p5, y3, p3)


def _q1_body(xyz_ref, g_ref, we_ref, ye_ref, se_ref, *, k):
    @pl.when(_first_step())
    def _():
        se_ref[...] = jnp.zeros_like(se_ref)

    q3 = xyz_ref[0][:, 0:3]
    ss = jnp.zeros((1, we_ref.shape[1]), F32)
    sq = jnp.zeros((1, we_ref.shape[1]), F32)
    for j in range(k):
        g = g_ref[0, j]
        gx = g[:, 64:67]
        dx = gx - q3
        euc = jnp.sqrt(jnp.sum(dx * dx, axis=1, keepdims=True) + 1e-20)
        xyzfeat = jnp.concatenate([q3, gx, dx, euc], axis=1)
        y = jnp.dot(xyzfeat, we_ref[...], preferred_element_type=F32)
        ye_ref[0, j] = y
        ss = ss + jnp.sum(y, axis=0, keepdims=True)
        sq = sq + jnp.sum(y * y, axis=0, keepdims=True)
    se_ref[0:1, :] = se_ref[0:1, :] + ss
    se_ref[1:2, :] = se_ref[1:2, :] + sq


def _q1_call(xyz8, g2, we):
    b, k, n, _ = g2.shape
    co = we.shape[1]
    return pl.pallas_call(
        functools.partial(_q1_body, k=k),
        grid=(b, n // _TN),
        in_specs=[
            pl.BlockSpec((1, _TN, 8), lambda bb, i: (bb, i, 0)),
            pl.BlockSpec((1, k, _TN, _TW), lambda bb, i: (bb, 0, i, 0)),
            pl.BlockSpec(we.shape, lambda bb, i: (0, 0)),
        ],
        out_specs=[
            pl.BlockSpec((1, k, _TN, co), lambda bb, i: (bb, 0, i, 0)),
            pl.BlockSpec((8, co), lambda bb, i: (0, 0)),
        ],
        out_shape=[
            jax.ShapeDtypeStruct((b, k, n, co), F32),
            jax.ShapeDtypeStruct((8, co), F32),
        ],
    )(xyz8, g2, we)


def _q2_body(ye_ref, pe_ref, wp_ref, g_ref, wa_ref, wb_ref, wc_ref,
             y6_ref, s6_ref, *, k):
    @pl.when(_first_step())
    def _():
        s6_ref[...] = jnp.zeros_like(s6_ref)

    esc, esh = pe_ref[0:1, :], pe_ref[1:2, :]
    wp = wp_ref[0]
    wpy = jnp.dot(wp, wb_ref[...], preferred_element_type=F32)
    ss = jnp.zeros((1, wa_ref.shape[1]), F32)
    sq = jnp.zeros((1, wa_ref.shape[1]), F32)
    for j in range(k):
        xe = jnp.maximum(ye_ref[0, j] * esc + esh, 0.0)
        gp = g_ref[0, j][:, 0:64]
        y = (jnp.dot(xe, wa_ref[...], preferred_element_type=F32)
             + wpy
             + jnp.dot(gp, wc_ref[...], preferred_element_type=F32))
        y6_ref[0, j] = y
        ss = ss + jnp.sum(y, axis=0, keepdims=True)
        sq = sq + jnp.sum(y * y, axis=0, keepdims=True)
    s6_ref[0:1, :] = s6_ref[0:1, :] + ss
    s6_ref[1:2, :] = s6_ref[1:2, :] + sq


def _q2_call(ye2, pe2, wpoints, g2, wa, wb, wc):
    b, k, n, ce = ye2.shape
    co = wa.shape[1]
    return pl.pallas_call(
        functools.partial(_q2_body, k=k),
        grid=(b, n // _TN),
        in_specs=[
            pl.BlockSpec((1, k, _TN, ce), lambda bb, i: (bb, 0, i, 0)),
            pl.BlockSpec((8, ce), lambda bb, i: (0, 0)),
            pl.BlockSpec((1, _TN, wpoints.shape[2]), lambda bb, i: (bb, i, 0)),
            pl.BlockSpec((1, k, _TN, _TW), lambda bb, i: (bb, 0, i, 0)),
            pl.BlockSpec(wa.shape, lambda bb, i: (0, 0)),
            pl.BlockSpec(wb.shape, lambda bb, i: (0, 0)),
            pl.BlockSpec(wc.shape, lambda bb, i: (0, 0)),
        ],
        out_specs=[
            pl.BlockSpec((1, k, _TN, co), lambda bb, i: (bb, 0, i, 0)),
            pl.BlockSpec((8, co), lambda bb, i: (0, 0)),
        ],
        out_shape=[
            jax.ShapeDtypeStruct((b, k, n, co), F32),
            jax.ShapeDtypeStruct((8, co), F32),
        ],
    )(ye2, pe2, wpoints, g2, wa, wb, wc)


def _q4_body(y7_ref, p7_ref, g_ref, out_ref, *, k):
    wsc, wsh = p7_ref[0:1, :], p7_ref[1:2, :]
    zs = [jnp.maximum(y7_ref[0, j] * wsc + wsh, 0.0) for j in range(k)]
    xs = [g_ref[0, j][:, 0:64] for j in range(k)]
    m = zs[0]
    for j in range(1, k):
        m = jnp.maximum(m, zs[j])
    es = [jnp.exp(z - m) for z in zs]
    tot = es[0]
    for j in range(1, k):
        tot = tot + es[j]
    acc = es[0] * xs[0]
    for j in range(1, k):
        acc = acc + es[j] * xs[j]
    out_ref[0] = acc / tot


def _q4_call(y7, p7, g2):
    b, k, n, c = y7.shape
    return pl.pallas_call(
        functools.partial(_q4_body, k=k),
        grid=(b, n // _TN),
        in_specs=[
            pl.BlockSpec((1, k, _TN, c), lambda bb, i: (bb, 0, i, 0)),
            pl.BlockSpec((8, c), lambda bb, i: (0, 0)),
            pl.BlockSpec((1, k, _TN, _TW), lambda bb, i: (bb, 0, i, 0)),
        ],
        out_specs=pl.BlockSpec((1, _TN, c), lambda bb, i: (bb, i, 0)),
        out_shape=jax.ShapeDtypeStruct((b, n, c), F32),
    )(y7, p7, g2)


def _ss(stats, gb, cnt):
    gamma, beta = gb
    mean = stats[0] / cnt
    var = stats[1] / cnt - mean * mean
    sc = gamma / jnp.sqrt(var + 1e-5)
    sh = beta - mean * sc
    return jnp.concatenate(
        [sc[None], sh[None], jnp.zeros((6,) + sc.shape, F32)], axis=0)


def kernel(warped_xyz, f2_xyz, warped_points, f2_points, params):
    b, n, _ = warped_xyz.shape
    m = f2_xyz.shape[1]

    pad_n = jnp.zeros((b, n, 5), F32)
    pad_m = jnp.zeros((b, m, 5), F32)
    wxyz8 = jnp.concatenate([warped_xyz, pad_n], axis=-1)
    fxyz8 = jnp.concatenate([f2_xyz, pad_m], axis=-1)
    fxyzT = jnp.swapaxes(fxyz8, 1, 2)
    wxyzT = jnp.swapaxes(wxyz8, 1, 2)

    idx1 = _knn_call(wxyz8, fxyzT, _K1)
    idx2 = _knn_call(wxyz8, wxyzT, _K2)
    idxq = jnp.swapaxes(idx1, 1, 2).reshape(-1)
    table1 = jnp.concatenate(
        [f2_points, fxyz8], axis=-1)
    table1 = jnp.concatenate(
        [table1, jnp.zeros((b, m, _TW - table1.shape[-1]), F32)],
        axis=-1).reshape(b * m, _TW)
    g1 = _gather_rows(table1, idxq).reshape(b, _K1, n, _TW)

    w1 = params['conv1'][0][0]
    y1, ye, s1, se = _p1_call(wxyz8, warped_points, g1,
                              w1[0:10], w1[10:74], w1[74:138],
                              params['enc1'][0])
    cnt1 = float(b * n * _K1)
    y2, s2 = _layer_call(y1, _ss(s1, params['bn1s'][0], cnt1),
                         params['conv1'][1][0])
    y3, s3 = _layer_call(y2, _ss(s2, params['bn1s'][1], cnt1),
                         params['conv1'][2][0])
    p3 = _ss(s3, params['bn1s'][2], cnt1)
    pe = _ss(se, params['bn_e1'], cnt1)
    w4 = params['conv2'][0][0]
    y4, s4 = _p4_call(ye, pe, y3, p3, w4[0:64], w4[64:128])
    y5, s5 = _layer_call(y4, _ss(s4, params['bn2s'][0], cnt1),
                         params['conv2'][1][0])
    p5 = _ss(s5, params['bn2s'][1], cnt1)
    feat1 = _p6_call(y5, p5, y3, p3)

    idxp = jnp.swapaxes(idx2, 1, 2).reshape(-1)
    table2 = jnp.concatenate(
        [feat1, wxyz8, jnp.zeros((b, n, _TW - 72), F32)],
        axis=-1).reshape(b * n, _TW)
    g2 = _gather_rows(table2, idxp).reshape(b, _K2, n, _TW)

    ye2, se2 = _q1_call(wxyz8, g2, params['enc2'][0])
    cnt2 = float(b * n * _K2)
    pe2 = _ss(se2, params['bn_e2'], cnt2)
    w6 = params['conv3'][0][0]
    y6, s6 = _q2_call(ye2, pe2, warped_points, g2,
                      w6[0:64], w6[64:128], w6[128:192])
    y7, s7 = _layer_call(y6, _ss(s6, params['bn2s'][0], cnt2),
                         params['conv3'][1][0])
    p7 = _ss(s7, params['bn2s'][1], cnt2)
    return _q4_call(y7, p7, g2)


# speedup vs baseline: 26.9936x; 1.0499x over previous
import functools

import jax
import jax.numpy as jnp
from jax import lax
from jax.experimental import pallas as pl
from jax.experimental.pallas import tpu as pltpu
from jax.experimental.pallas import tpu_sc as plsc

F32 = jnp.float32
_TN = 1024
_K1 = 6
_K2 = 4
_TW = 128
_NW = 32
_CHUNK = 128


def _first_step():
    return (pl.program_id(0) == 0) & (pl.program_id(1) == 0)


def _knn_body(q_ref, xt_ref, idx_ref, *, k, m):
    q = q_ref[0]
    xt = xt_ref[0]
    d = -2.0 * jnp.dot(q, xt, preferred_element_type=F32)
    d = d + jnp.sum(q * q, axis=1, keepdims=True)
    d = d + jnp.sum(xt * xt, axis=0, keepdims=True)
    iota = lax.broadcasted_iota(jnp.int32, d.shape, 1)
    cols = []
    for _ in range(k):
        am = jnp.argmin(d, axis=1)[:, None]
        cols.append(am)
        d = jnp.where(iota == am, jnp.inf, d)
    idx_ref[0] = jnp.concatenate(cols, axis=1) + pl.program_id(0) * m


def _knn_call(q8, xt8, k):
    b, n, _ = q8.shape
    m = xt8.shape[2]
    return pl.pallas_call(
        functools.partial(_knn_body, k=k, m=m),
        grid=(b, n // _TN),
        in_specs=[
            pl.BlockSpec((1, _TN, 8), lambda bb, i: (bb, i, 0)),
            pl.BlockSpec((1, 8, m), lambda bb, i: (bb, 0, 0)),
        ],
        out_specs=pl.BlockSpec((1, _TN, k), lambda bb, i: (bb, i, 0)),
        out_shape=jax.ShapeDtypeStruct((b, n, k), jnp.int32),
    )(q8, xt8)


def _gather_rows(table, idx):
    rows = idx.shape[0]
    per_w = rows // _NW
    chunks = per_w // _CHUNK
    idx3 = idx.reshape(_NW, chunks, _CHUNK)
    mesh = plsc.VectorSubcoreMesh(core_axis_name="c", subcore_axis_name="s",
                                  num_cores=2)

    @functools.partial(
        pl.kernel, mesh=mesh,
        out_type=jax.ShapeDtypeStruct((rows, _TW), F32),
        scratch_types=[
            pltpu.VMEM((chunks, _CHUNK), jnp.int32),
            pltpu.VMEM((2, _CHUNK, _TW), F32),
            pltpu.SemaphoreType.DMA,
            pltpu.SemaphoreType.DMA,
        ],
    )
    def gk(table_hbm, idx_hbm, out_hbm, idx_v, rows_v, sem0, sem1):
        wid = lax.axis_index("s") * 2 + lax.axis_index("c")
        base = wid * per_w
        pltpu.sync_copy(idx_hbm.at[wid], idx_v)
        sems = (sem0, sem1)
        prev = None
        for j in range(chunks):
            cp = pltpu.async_copy(table_hbm.at[idx_v.at[j]],
                                  rows_v.at[j % 2], sems[j % 2])
            if prev is not None:
                pj, pcp = prev
                pcp.wait()
                pltpu.sync_copy(
                    rows_v.at[pj % 2],
                    out_hbm.at[pl.ds(base + pj * _CHUNK, _CHUNK)])
            prev = (j, cp)
        pj, pcp = prev
        pcp.wait()
        pltpu.sync_copy(rows_v.at[pj % 2],
                        out_hbm.at[pl.ds(base + pj * _CHUNK, _CHUNK)])

    return gk(table, idx3)


def _p1_body(xyz_ref, wp_ref, g_ref, w1a_ref, w1b_ref, w1c_ref, we_ref,
             y1_ref, ye_ref, s1_ref, se_ref):
    @pl.when(_first_step())
    def _():
        s1_ref[...] = jnp.zeros_like(s1_ref)
        se_ref[...] = jnp.zeros_like(se_ref)

    q3 = xyz_ref[0][:, 0:3]
    wp = wp_ref[0]
    s1s = jnp.zeros((1, y1_ref.shape[-1]), F32)
    s1q = jnp.zeros((1, y1_ref.shape[-1]), F32)
    ses = jnp.zeros((1, ye_ref.shape[-1]), F32)
    seq = jnp.zeros((1, ye_ref.shape[-1]), F32)
    for j in range(_K1):
        g = g_ref[0, j]
        gp = g[:, 0:64]
        gx = g[:, 64:67]
        dx = gx - q3
        euc = jnp.sqrt(jnp.sum(dx * dx, axis=1, keepdims=True) + 1e-20)
        xyzfeat = jnp.concatenate([q3, gx, dx, euc], axis=1)
        y = (jnp.dot(xyzfeat, w1a_ref[...], preferred_element_type=F32)
             + jnp.dot(wp, w1b_ref[...], preferred_element_type=F32)
             + jnp.dot(gp, w1c_ref[...], preferred_element_type=F32))
        ye = jnp.dot(xyzfeat, we_ref[...], preferred_element_type=F32)
        y1_ref[0, j] = y
        ye_ref[0, j] = ye
        s1s = s1s + jnp.sum(y, axis=0, keepdims=True)
        s1q = s1q + jnp.sum(y * y, axis=0, keepdims=True)
        ses = ses + jnp.sum(ye, axis=0, keepdims=True)
        seq = seq + jnp.sum(ye * ye, axis=0, keepdims=True)
    s1_ref[0:1, :] = s1_ref[0:1, :] + s1s
    s1_ref[1:2, :] = s1_ref[1:2, :] + s1q
    se_ref[0:1, :] = se_ref[0:1, :] + ses
    se_ref[1:2, :] = se_ref[1:2, :] + seq


def _p1_call(xyz8, wpoints, g1, w1a, w1b, w1c, we):
    b, k, n, _ = g1.shape
    co, ce = w1a.shape[1], we.shape[1]
    return pl.pallas_call(
        _p1_body,
        grid=(b, n // _TN),
        in_specs=[
            pl.BlockSpec((1, _TN, 8), lambda bb, i: (bb, i, 0)),
            pl.BlockSpec((1, _TN, wpoints.shape[2]), lambda bb, i: (bb, i, 0)),
            pl.BlockSpec((1, k, _TN, _TW), lambda bb, i: (bb, 0, i, 0)),
            pl.BlockSpec(w1a.shape, lambda bb, i: (0, 0)),
            pl.BlockSpec(w1b.shape, lambda bb, i: (0, 0)),
            pl.BlockSpec(w1c.shape, lambda bb, i: (0, 0)),
            pl.BlockSpec(we.shape, lambda bb, i: (0, 0)),
        ],
        out_specs=[
            pl.BlockSpec((1, k, _TN, co), lambda bb, i: (bb, 0, i, 0)),
            pl.BlockSpec((1, k, _TN, ce), lambda bb, i: (bb, 0, i, 0)),
            pl.BlockSpec((8, co), lambda bb, i: (0, 0)),
            pl.BlockSpec((8, ce), lambda bb, i: (0, 0)),
        ],
        out_shape=[
            jax.ShapeDtypeStruct((b, k, n, co), F32),
            jax.ShapeDtypeStruct((b, k, n, ce), F32),
            jax.ShapeDtypeStruct((8, co), F32),
            jax.ShapeDtypeStruct((8, ce), F32),
        ],
    )(xyz8, wpoints, g1, w1a, w1b, w1c, we)


def _layer_body(yin_ref, p_ref, w_ref, yout_ref, s_ref, *, k):
    @pl.when(_first_step())
    def _():
        s_ref[...] = jnp.zeros_like(s_ref)

    sc = p_ref[0:1, :]
    sh = p_ref[1:2, :]
    ss = jnp.zeros((1, w_ref.shape[1]), F32)
    sq = jnp.zeros((1, w_ref.shape[1]), F32)
    for j in range(k):
        x = jnp.maximum(yin_ref[0, j] * sc + sh, 0.0)
        y = jnp.dot(x, w_ref[...], preferred_element_type=F32)
        yout_ref[0, j] = y
        ss = ss + jnp.sum(y, axis=0, keepdims=True)
        sq = sq + jnp.sum(y * y, axis=0, keepdims=True)
    s_ref[0:1, :] = s_ref[0:1, :] + ss
    s_ref[1:2, :] = s_ref[1:2, :] + sq


def _layer_call(yin, p, w):
    b, k, n, cin = yin.shape
    co = w.shape[1]
    return pl.pallas_call(
        functools.partial(_layer_body, k=k),
        grid=(b, n // _TN),
        in_specs=[
            pl.BlockSpec((1, k, _TN, cin), lambda bb, i: (bb, 0, i, 0)),
            pl.BlockSpec((8, cin), lambda bb, i: (0, 0)),
            pl.BlockSpec(w.shape, lambda bb, i: (0, 0)),
        ],
        out_specs=[
            pl.BlockSpec((1, k, _TN, co), lambda bb, i: (bb, 0, i, 0)),
            pl.BlockSpec((8, co), lambda bb, i: (0, 0)),
        ],
        out_shape=[
            jax.ShapeDtypeStruct((b, k, n, co), F32),
            jax.ShapeDtypeStruct((8, co), F32),
        ],
    )(yin, p, w)


def _p4_body(ye_ref, pe_ref, y3_ref, p3_ref, wa_ref, wb_ref,
             y4_ref, s4_ref, *, k):
    @pl.when(_first_step())
    def _():
        s4_ref[...] = jnp.zeros_like(s4_ref)

    esc, esh = pe_ref[0:1, :], pe_ref[1:2, :]
    xsc, xsh = p3_ref[0:1, :], p3_ref[1:2, :]
    ss = jnp.zeros((1, wa_ref.shape[1]), F32)
    sq = jnp.zeros((1, wa_ref.shape[1]), F32)
    for j in range(k):
        xe = jnp.maximum(ye_ref[0, j] * esc + esh, 0.0)
        x3 = jnp.maximum(y3_ref[0, j] * xsc + xsh, 0.0)
        y = (jnp.dot(xe, wa_ref[...], preferred_element_type=F32)
             + jnp.dot(x3, wb_ref[...], preferred_element_type=F32))
        y4_ref[0, j] = y
        ss = ss + jnp.sum(y, axis=0, keepdims=True)
        sq = sq + jnp.sum(y * y, axis=0, keepdims=True)
    s4_ref[0:1, :] = s4_ref[0:1, :] + ss
    s4_ref[1:2, :] = s4_ref[1:2, :] + sq


def _p4_call(ye, pe, y3, p3, wa, wb):
    b, k, n, ce = ye.shape
    co = wa.shape[1]
    return pl.pallas_call(
        functools.partial(_p4_body, k=k),
        grid=(b, n // _TN),
        in_specs=[
            pl.BlockSpec((1, k, _TN, ce), lambda bb, i: (bb, 0, i, 0)),
            pl.BlockSpec((8, ce), lambda bb, i: (0, 0)),
            pl.BlockSpec((1, k, _TN, y3.shape[3]), lambda bb, i: (bb, 0, i, 0)),
            pl.BlockSpec((8, y3.shape[3]), lambda bb, i: (0, 0)),
            pl.BlockSpec(wa.shape, lambda bb, i: (0, 0)),
            pl.BlockSpec(wb.shape, lambda bb, i: (0, 0)),
        ],
        out_specs=[
            pl.BlockSpec((1, k, _TN, co), lambda bb, i: (bb, 0, i, 0)),
            pl.BlockSpec((8, co), lambda bb, i: (0, 0)),
        ],
        out_shape=[
            jax.ShapeDtypeStruct((b, k, n, co), F32),
            jax.ShapeDtypeStruct((8, co), F32),
        ],
    )(ye, pe, y3, p3, wa, wb)


def _p6_body(y5_ref, p5_ref, y3_ref, p3_ref, out_ref, *, k):
    wsc, wsh = p5_ref[0:1, :], p5_ref[1:2, :]
    xsc, xsh = p3_ref[0:1, :], p3_ref[1:2, :]
    zs = [jnp.maximum(y5_ref[0, j] * wsc + wsh, 0.0) for j in range(k)]
    xs = [jnp.maximum(y3_ref[0, j] * xsc + xsh, 0.0) for j in range(k)]
    m = zs[0]
    for j in range(1, k):
        m = jnp.maximum(m, zs[j])
    es = [jnp.exp(z - m) for z in zs]
    tot = es[0]
    for j in range(1, k):
        tot = tot + es[j]
    acc = es[0] * xs[0]
    for j in range(1, k):
        acc = acc + es[j] * xs[j]
    out_ref[0] = acc / tot


def _p6_call(y5, p5, y3, p3):
    b, k, n, c = y5.shape
    return pl.pallas_call(
        functools.partial(_p6_body, k=k),
        grid=(b, n // _TN),
        in_specs=[
            pl.BlockSpec((1, k, _TN, c), lambda bb, i: (bb, 0, i, 0)),
            pl.BlockSpec((8, c), lambda bb, i: (0, 0)),
            pl.BlockSpec((1, k, _TN, y3.shape[3]), lambda bb, i: (bb, 0, i, 0)),
            pl.BlockSpec((8, y3.shape[3]), lambda bb, i: (0, 0)),
        ],
        out_specs=pl.BlockSpec((1, _TN, y3.shape[3]), lambda bb, i: (bb, i, 0)),
        out_shape=jax.ShapeDtypeStruct((b, n, y3.shape[3]), F32),
    )(y5, p5, y3, p3)


def _q1_body(xyz_ref, g_ref, we_ref, ye_ref, se_ref, *, k):
    @pl.when(_first_step())
    def _():
        se_ref[...] = jnp.zeros_like(se_ref)

    q3 = xyz_ref[0][:, 0:3]
    ss = jnp.zeros((1, we_ref.shape[1]), F32)
    sq = jnp.zeros((1, we_ref.shape[1]), F32)
    for j in range(k):
        g = g_ref[0, j]
        gx = g[:, 64:67]
        dx = gx - q3
        euc = jnp.sqrt(jnp.sum(dx * dx, axis=1, keepdims=True) + 1e-20)
        xyzfeat = jnp.concatenate([q3, gx, dx, euc], axis=1)
        y = jnp.dot(xyzfeat, we_ref[...], preferred_element_type=F32)
        ye_ref[0, j] = y
        ss = ss + jnp.sum(y, axis=0, keepdims=True)
        sq = sq + jnp.sum(y * y, axis=0, keepdims=True)
    se_ref[0:1, :] = se_ref[0:1, :] + ss
    se_ref[1:2, :] = se_ref[1:2, :] + sq


def _q1_call(xyz8, g2, we):
    b, k, n, _ = g2.shape
    co = we.shape[1]
    return pl.pallas_call(
        functools.partial(_q1_body, k=k),
        grid=(b, n // _TN),
        in_specs=[
            pl.BlockSpec((1, _TN, 8), lambda bb, i: (bb, i, 0)),
            pl.BlockSpec((1, k, _TN, _TW), lambda bb, i: (bb, 0, i, 0)),
            pl.BlockSpec(we.shape, lambda bb, i: (0, 0)),
        ],
        out_specs=[
            pl.BlockSpec((1, k, _TN, co), lambda bb, i: (bb, 0, i, 0)),
            pl.BlockSpec((8, co), lambda bb, i: (0, 0)),
        ],
        out_shape=[
            jax.ShapeDtypeStruct((b, k, n, co), F32),
            jax.ShapeDtypeStruct((8, co), F32),
        ],
    )(xyz8, g2, we)


def _q2_body(ye_ref, pe_ref, wp_ref, g_ref, wa_ref, wb_ref, wc_ref,
             y6_ref, s6_ref, *, k):
    @pl.when(_first_step())
    def _():
        s6_ref[...] = jnp.zeros_like(s6_ref)

    esc, esh = pe_ref[0:1, :], pe_ref[1:2, :]
    wp = wp_ref[0]
    wpy = jnp.dot(wp, wb_ref[...], preferred_element_type=F32)
    ss = jnp.zeros((1, wa_ref.shape[1]), F32)
    sq = jnp.zeros((1, wa_ref.shape[1]), F32)
    for j in range(k):
        xe = jnp.maximum(ye_ref[0, j] * esc + esh, 0.0)
        gp = g_ref[0, j][:, 0:64]
        y = (jnp.dot(xe, wa_ref[...], preferred_element_type=F32)
             + wpy
             + jnp.dot(gp, wc_ref[...], preferred_element_type=F32))
        y6_ref[0, j] = y
        ss = ss + jnp.sum(y, axis=0, keepdims=True)
        sq = sq + jnp.sum(y * y, axis=0, keepdims=True)
    s6_ref[0:1, :] = s6_ref[0:1, :] + ss
    s6_ref[1:2, :] = s6_ref[1:2, :] + sq


def _q2_call(ye2, pe2, wpoints, g2, wa, wb, wc):
    b, k, n, ce = ye2.shape
    co = wa.shape[1]
    return pl.pallas_call(
        functools.partial(_q2_body, k=k),
        grid=(b, n // _TN),
        in_specs=[
            pl.BlockSpec((1, k, _TN, ce), lambda bb, i: (bb, 0, i, 0)),
            pl.BlockSpec((8, ce), lambda bb, i: (0, 0)),
            pl.BlockSpec((1, _TN, wpoints.shape[2]), lambda bb, i: (bb, i, 0)),
            pl.BlockSpec((1, k, _TN, _TW), lambda bb, i: (bb, 0, i, 0)),
            pl.BlockSpec(wa.shape, lambda bb, i: (0, 0)),
            pl.BlockSpec(wb.shape, lambda bb, i: (0, 0)),
            pl.BlockSpec(wc.shape, lambda bb, i: (0, 0)),
        ],
        out_specs=[
            pl.BlockSpec((1, k, _TN, co), lambda bb, i: (bb, 0, i, 0)),
            pl.BlockSpec((8, co), lambda bb, i: (0, 0)),
        ],
        out_shape=[
            jax.ShapeDtypeStruct((b, k, n, co), F32),
            jax.ShapeDtypeStruct((8, co), F32),
        ],
    )(ye2, pe2, wpoints, g2, wa, wb, wc)


def _q4_body(y7_ref, p7_ref, g_ref, out_ref, *, k):
    wsc, wsh = p7_ref[0:1, :], p7_ref[1:2, :]
    zs = [jnp.maximum(y7_ref[0, j] * wsc + wsh, 0.0) for j in range(k)]
    xs = [g_ref[0, j][:, 0:64] for j in range(k)]
    m = zs[0]
    for j in range(1, k):
        m = jnp.maximum(m, zs[j])
    es = [jnp.exp(z - m) for z in zs]
    tot = es[0]
    for j in range(1, k):
        tot = tot + es[j]
    acc = es[0] * xs[0]
    for j in range(1, k):
        acc = acc + es[j] * xs[j]
    out_ref[0] = acc / tot


def _q4_call(y7, p7, g2):
    b, k, n, c = y7.shape
    return pl.pallas_call(
        functools.partial(_q4_body, k=k),
        grid=(b, n // _TN),
        in_specs=[
            pl.BlockSpec((1, k, _TN, c), lambda bb, i: (bb, 0, i, 0)),
            pl.BlockSpec((8, c), lambda bb, i: (0, 0)),
            pl.BlockSpec((1, k, _TN, _TW), lambda bb, i: (bb, 0, i, 0)),
        ],
        out_specs=pl.BlockSpec((1, _TN, c), lambda bb, i: (bb, i, 0)),
        out_shape=jax.ShapeDtypeStruct((b, n, c), F32),
    )(y7, p7, g2)


def _ss(stats, gb, cnt):
    gamma, beta = gb
    mean = stats[0] / cnt
    var = stats[1] / cnt - mean * mean
    sc = gamma / jnp.sqrt(var + 1e-5)
    sh = beta - mean * sc
    return jnp.concatenate(
        [sc[None], sh[None], jnp.zeros((6,) + sc.shape, F32)], axis=0)


def kernel(warped_xyz, f2_xyz, warped_points, f2_points, params):
    b, n, _ = warped_xyz.shape
    m = f2_xyz.shape[1]

    pad_n = jnp.zeros((b, n, 5), F32)
    pad_m = jnp.zeros((b, m, 5), F32)
    wxyz8 = jnp.concatenate([warped_xyz, pad_n], axis=-1)
    fxyz8 = jnp.concatenate([f2_xyz, pad_m], axis=-1)
    fxyzT = jnp.swapaxes(fxyz8, 1, 2)
    wxyzT = jnp.swapaxes(wxyz8, 1, 2)

    idx1 = _knn_call(wxyz8, fxyzT, _K1)
    idx2 = _knn_call(wxyz8, wxyzT, _K2)
    idxq = jnp.swapaxes(idx1, 1, 2).reshape(-1)
    table1 = jnp.concatenate(
        [f2_points, fxyz8], axis=-1)
    table1 = jnp.concatenate(
        [table1, jnp.zeros((b, m, _TW - table1.shape[-1]), F32)],
        axis=-1).reshape(b * m, _TW)
    g1 = _gather_rows(table1, idxq).reshape(b, _K1, n, _TW)

    w1 = params['conv1'][0][0]
    y1, ye, s1, se = _p1_call(wxyz8, warped_points, g1,
                              w1[0:10], w1[10:74], w1[74:138],
                              params['enc1'][0])
    cnt1 = float(b * n * _K1)
    y2, s2 = _layer_call(y1, _ss(s1, params['bn1s'][0], cnt1),
                         params['conv1'][1][0])
    y3, s3 = _layer_call(y2, _ss(s2, params['bn1s'][1], cnt1),
                         params['conv1'][2][0])
    p3 = _ss(s3, params['bn1s'][2], cnt1)
    pe = _ss(se, params['bn_e1'], cnt1)
    w4 = params['conv2'][0][0]
    y4, s4 = _p4_call(ye, pe, y3, p3, w4[0:64], w4[64:128])
    y5, s5 = _layer_call(y4, _ss(s4, params['bn2s'][0], cnt1),
                         params['conv2'][1][0])
    p5 = _ss(s5, params['bn2s'][1], cnt1)
    feat1 = _p6_call(y5, p5, y3, p3)

    idxp = jnp.swapaxes(idx2, 1, 2).reshape(-1)
    table2 = jnp.concatenate(
        [feat1, wxyz8, jnp.zeros((b, n, _TW - 72), F32)],
        axis=-1).reshape(b * n, _TW)
    g2 = _gather_rows(table2, idxp).reshape(b, _K2, n, _TW)

    ye2, se2 = _q1_call(wxyz8, g2, params['enc2'][0])
    cnt2 = float(b * n * _K2)
    pe2 = _ss(se2, params['bn_e2'], cnt2)
    w6 = params['conv3'][0][0]
    y6, s6 = _q2_call(ye2, pe2, warped_points, g2,
                      w6[0:64], w6[64:128], w6[128:192])
    y7, s7 = _layer_call(y6, _ss(s6, params['bn2s'][0], cnt2),
                         params['conv3'][1][0])
    p7 = _ss(s7, params['bn2s'][1], cnt2)
    return _q4_call(y7, p7, g2)


# speedup vs baseline: 27.3202x; 1.0121x over previous
import functools

import jax
import jax.numpy as jnp
from jax import lax
from jax.experimental import pallas as pl
from jax.experimental.pallas import tpu as pltpu
from jax.experimental.pallas import tpu_sc as plsc

F32 = jnp.float32
_TN = 1024
_K1 = 6
_K2 = 4
_TW = 128
_NW = 32
_CHUNK = 128


def _first_step():
    return (pl.program_id(0) == 0) & (pl.program_id(1) == 0)


_TNK = 512


def _topk_idx(q, xt, k, m):
    d = -2.0 * jnp.dot(q, xt, preferred_element_type=F32)
    d = d + jnp.sum(q * q, axis=1, keepdims=True)
    d = d + jnp.sum(xt * xt, axis=0, keepdims=True)
    iota = lax.broadcasted_iota(jnp.int32, d.shape, 1)
    cols = []
    for _ in range(k):
        am = jnp.argmin(d, axis=1)[:, None]
        cols.append(am)
        d = jnp.where(iota == am, jnp.inf, d)
    return jnp.concatenate(cols, axis=1) + pl.program_id(0) * m


def _knn_body(q_ref, fxt_ref, wxt_ref, fp_ref, fx_ref,
              idx1_ref, idx2_ref, tab_ref, *, m):
    q = q_ref[0]
    idx1_ref[0] = _topk_idx(q, fxt_ref[0], _K1, m)
    idx2_ref[0] = _topk_idx(q, wxt_ref[0], _K2, m)
    fp = fp_ref[0]
    fx = fx_ref[0]
    pad = jnp.zeros((fp.shape[0], _TW - 72), F32)
    tab_ref[...] = jnp.concatenate([fp, fx, pad], axis=1)


def _knn_call(wxyz8, fxyzT, wxyzT, f2_points, fxyz8):
    b, n, _ = wxyz8.shape
    m = fxyzT.shape[2]
    nt = n // _TNK
    return pl.pallas_call(
        functools.partial(_knn_body, m=m),
        grid=(b, nt),
        in_specs=[
            pl.BlockSpec((1, _TNK, 8), lambda bb, i: (bb, i, 0)),
            pl.BlockSpec((1, 8, m), lambda bb, i: (bb, 0, 0)),
            pl.BlockSpec((1, 8, m), lambda bb, i: (bb, 0, 0)),
            pl.BlockSpec((1, _TNK, 64), lambda bb, i: (bb, i, 0)),
            pl.BlockSpec((1, _TNK, 8), lambda bb, i: (bb, i, 0)),
        ],
        out_specs=[
            pl.BlockSpec((1, _TNK, _K1), lambda bb, i: (bb, i, 0)),
            pl.BlockSpec((1, _TNK, _K2), lambda bb, i: (bb, i, 0)),
            pl.BlockSpec((_TNK, _TW), lambda bb, i, _nt=nt: (bb * _nt + i, 0)),
        ],
        out_shape=[
            jax.ShapeDtypeStruct((b, n, _K1), jnp.int32),
            jax.ShapeDtypeStruct((b, n, _K2), jnp.int32),
            jax.ShapeDtypeStruct((b * m, _TW), F32),
        ],
    )(wxyz8, fxyzT, wxyzT, f2_points, fxyz8)


def _gather_rows(table, idx):
    rows = idx.shape[0]
    per_w = rows // _NW
    chunks = per_w // _CHUNK
    idx3 = idx.reshape(_NW, chunks, _CHUNK)
    mesh = plsc.VectorSubcoreMesh(core_axis_name="c", subcore_axis_name="s",
                                  num_cores=2)

    @functools.partial(
        pl.kernel, mesh=mesh,
        out_type=jax.ShapeDtypeStruct((rows, _TW), F32),
        scratch_types=[
            pltpu.VMEM((chunks, _CHUNK), jnp.int32),
            pltpu.VMEM((2, _CHUNK, _TW), F32),
            pltpu.SemaphoreType.DMA,
            pltpu.SemaphoreType.DMA,
        ],
    )
    def gk(table_hbm, idx_hbm, out_hbm, idx_v, rows_v, sem0, sem1):
        wid = lax.axis_index("s") * 2 + lax.axis_index("c")
        base = wid * per_w
        pltpu.sync_copy(idx_hbm.at[wid], idx_v)
        sems = (sem0, sem1)
        prev = None
        for j in range(chunks):
            cp = pltpu.async_copy(table_hbm.at[idx_v.at[j]],
                                  rows_v.at[j % 2], sems[j % 2])
            if prev is not None:
                pj, pcp = prev
                pcp.wait()
                pltpu.sync_copy(
                    rows_v.at[pj % 2],
                    out_hbm.at[pl.ds(base + pj * _CHUNK, _CHUNK)])
            prev = (j, cp)
        pj, pcp = prev
        pcp.wait()
        pltpu.sync_copy(rows_v.at[pj % 2],
                        out_hbm.at[pl.ds(base + pj * _CHUNK, _CHUNK)])

    return gk(table, idx3)


def _p1_body(xyz_ref, wp_ref, g_ref, w1a_ref, w1b_ref, w1c_ref, we_ref,
             y1_ref, ye_ref, s1_ref, se_ref):
    @pl.when(_first_step())
    def _():
        s1_ref[...] = jnp.zeros_like(s1_ref)
        se_ref[...] = jnp.zeros_like(se_ref)

    q3 = xyz_ref[0][:, 0:3]
    wp = wp_ref[0]
    s1s = jnp.zeros((1, y1_ref.shape[-1]), F32)
    s1q = jnp.zeros((1, y1_ref.shape[-1]), F32)
    ses = jnp.zeros((1, ye_ref.shape[-1]), F32)
    seq = jnp.zeros((1, ye_ref.shape[-1]), F32)
    for j in range(_K1):
        g = g_ref[0, j]
        gp = g[:, 0:64]
        gx = g[:, 64:67]
        dx = gx - q3
        euc = jnp.sqrt(jnp.sum(dx * dx, axis=1, keepdims=True) + 1e-20)
        xyzfeat = jnp.concatenate([q3, gx, dx, euc], axis=1)
        y = (jnp.dot(xyzfeat, w1a_ref[...], preferred_element_type=F32)
             + jnp.dot(wp, w1b_ref[...], preferred_element_type=F32)
             + jnp.dot(gp, w1c_ref[...], preferred_element_type=F32))
        ye = jnp.dot(xyzfeat, we_ref[...], preferred_element_type=F32)
        y1_ref[0, j] = y
        ye_ref[0, j] = ye
        s1s = s1s + jnp.sum(y, axis=0, keepdims=True)
        s1q = s1q + jnp.sum(y * y, axis=0, keepdims=True)
        ses = ses + jnp.sum(ye, axis=0, keepdims=True)
        seq = seq + jnp.sum(ye * ye, axis=0, keepdims=True)
    s1_ref[0:1, :] = s1_ref[0:1, :] + s1s
    s1_ref[1:2, :] = s1_ref[1:2, :] + s1q
    se_ref[0:1, :] = se_ref[0:1, :] + ses
    se_ref[1:2, :] = se_ref[1:2, :] + seq


def _p1_call(xyz8, wpoints, g1, w1a, w1b, w1c, we):
    b, k, n, _ = g1.shape
    co, ce = w1a.shape[1], we.shape[1]
    return pl.pallas_call(
        _p1_body,
        grid=(b, n // _TN),
        in_specs=[
            pl.BlockSpec((1, _TN, 8), lambda bb, i: (bb, i, 0)),
            pl.BlockSpec((1, _TN, wpoints.shape[2]), lambda bb, i: (bb, i, 0)),
            pl.BlockSpec((1, k, _TN, _TW), lambda bb, i: (bb, 0, i, 0)),
            pl.BlockSpec(w1a.shape, lambda bb, i: (0, 0)),
            pl.BlockSpec(w1b.shape, lambda bb, i: (0, 0)),
            pl.BlockSpec(w1c.shape, lambda bb, i: (0, 0)),
            pl.BlockSpec(we.shape, lambda bb, i: (0, 0)),
        ],
        out_specs=[
            pl.BlockSpec((1, k, _TN, co), lambda bb, i: (bb, 0, i, 0)),
            pl.BlockSpec((1, k, _TN, ce), lambda bb, i: (bb, 0, i, 0)),
            pl.BlockSpec((8, co), lambda bb, i: (0, 0)),
            pl.BlockSpec((8, ce), lambda bb, i: (0, 0)),
        ],
        out_shape=[
            jax.ShapeDtypeStruct((b, k, n, co), F32),
            jax.ShapeDtypeStruct((b, k, n, ce), F32),
            jax.ShapeDtypeStruct((8, co), F32),
            jax.ShapeDtypeStruct((8, ce), F32),
        ],
    )(xyz8, wpoints, g1, w1a, w1b, w1c, we)


def _bn_fold(s_ref, gb_ref, cnt):
    mean = s_ref[0:1, :] / cnt
    var = s_ref[1:2, :] / cnt - mean * mean
    sc = gb_ref[0:1, :] / jnp.sqrt(var + 1e-5)
    sh = gb_ref[1:2, :] - mean * sc
    return sc, sh


def _layer_body(yin_ref, sin_ref, gb_ref, w_ref, yout_ref, s_ref, *, k, cnt):
    @pl.when(_first_step())
    def _():
        s_ref[...] = jnp.zeros_like(s_ref)

    sc, sh = _bn_fold(sin_ref, gb_ref, cnt)
    ss = jnp.zeros((1, w_ref.shape[1]), F32)
    sq = jnp.zeros((1, w_ref.shape[1]), F32)
    for j in range(k):
        x = jnp.maximum(yin_ref[0, j] * sc + sh, 0.0)
        y = jnp.dot(x, w_ref[...], preferred_element_type=F32)
        yout_ref[0, j] = y
        ss = ss + jnp.sum(y, axis=0, keepdims=True)
        sq = sq + jnp.sum(y * y, axis=0, keepdims=True)
    s_ref[0:1, :] = s_ref[0:1, :] + ss
    s_ref[1:2, :] = s_ref[1:2, :] + sq


def _layer_call(yin, sin, gb, w, cnt):
    b, k, n, cin = yin.shape
    co = w.shape[1]
    return pl.pallas_call(
        functools.partial(_layer_body, k=k, cnt=cnt),
        grid=(b, n // _TN),
        in_specs=[
            pl.BlockSpec((1, k, _TN, cin), lambda bb, i: (bb, 0, i, 0)),
            pl.BlockSpec((8, cin), lambda bb, i: (0, 0)),
            pl.BlockSpec((8, cin), lambda bb, i: (0, 0)),
            pl.BlockSpec(w.shape, lambda bb, i: (0, 0)),
        ],
        out_specs=[
            pl.BlockSpec((1, k, _TN, co), lambda bb, i: (bb, 0, i, 0)),
            pl.BlockSpec((8, co), lambda bb, i: (0, 0)),
        ],
        out_shape=[
            jax.ShapeDtypeStruct((b, k, n, co), F32),
            jax.ShapeDtypeStruct((8, co), F32),
        ],
    )(yin, sin, gb, w)


def _p4_body(ye_ref, se_ref, gbe_ref, y3_ref, s3_ref, gb3_ref, wa_ref, wb_ref,
             y4_ref, s4_ref, *, k, cnt):
    @pl.when(_first_step())
    def _():
        s4_ref[...] = jnp.zeros_like(s4_ref)

    esc, esh = _bn_fold(se_ref, gbe_ref, cnt)
    xsc, xsh = _bn_fold(s3_ref, gb3_ref, cnt)
    ss = jnp.zeros((1, wa_ref.shape[1]), F32)
    sq = jnp.zeros((1, wa_ref.shape[1]), F32)
    for j in range(k):
        xe = jnp.maximum(ye_ref[0, j] * esc + esh, 0.0)
        x3 = jnp.maximum(y3_ref[0, j] * xsc + xsh, 0.0)
        y = (jnp.dot(xe, wa_ref[...], preferred_element_type=F32)
             + jnp.dot(x3, wb_ref[...], preferred_element_type=F32))
        y4_ref[0, j] = y
        ss = ss + jnp.sum(y, axis=0, keepdims=True)
        sq = sq + jnp.sum(y * y, axis=0, keepdims=True)
    s4_ref[0:1, :] = s4_ref[0:1, :] + ss
    s4_ref[1:2, :] = s4_ref[1:2, :] + sq


def _p4_call(ye, se, gbe, y3, s3, gb3, wa, wb, cnt):
    b, k, n, ce = ye.shape
    co = wa.shape[1]
    return pl.pallas_call(
        functools.partial(_p4_body, k=k, cnt=cnt),
        grid=(b, n // _TN),
        in_specs=[
            pl.BlockSpec((1, k, _TN, ce), lambda bb, i: (bb, 0, i, 0)),
            pl.BlockSpec((8, ce), lambda bb, i: (0, 0)),
            pl.BlockSpec((8, ce), lambda bb, i: (0, 0)),
            pl.BlockSpec((1, k, _TN, y3.shape[3]), lambda bb, i: (bb, 0, i, 0)),
            pl.BlockSpec((8, y3.shape[3]), lambda bb, i: (0, 0)),
            pl.BlockSpec((8, y3.shape[3]), lambda bb, i: (0, 0)),
            pl.BlockSpec(wa.shape, lambda bb, i: (0, 0)),
            pl.BlockSpec(wb.shape, lambda bb, i: (0, 0)),
        ],
        out_specs=[
            pl.BlockSpec((1, k, _TN, co), lambda bb, i: (bb, 0, i, 0)),
            pl.BlockSpec((8, co), lambda bb, i: (0, 0)),
        ],
        out_shape=[
            jax.ShapeDtypeStruct((b, k, n, co), F32),
            jax.ShapeDtypeStruct((8, co), F32),
        ],
    )(ye, se, gbe, y3, s3, gb3, wa, wb)


def _p6_body(y5_ref, s5_ref, gb5_ref, y3_ref, s3_ref, gb3_ref, out_ref,
             *, k, cnt):
    wsc, wsh = _bn_fold(s5_ref, gb5_ref, cnt)
    xsc, xsh = _bn_fold(s3_ref, gb3_ref, cnt)
    zs = [jnp.maximum(y5_ref[0, j] * wsc + wsh, 0.0) for j in range(k)]
    xs = [jnp.maximum(y3_ref[0, j] * xsc + xsh, 0.0) for j in range(k)]
    m = zs[0]
    for j in range(1, k):
        m = jnp.maximum(m, zs[j])
    es = [jnp.exp(z - m) for z in zs]
    tot = es[0]
    for j in range(1, k):
        tot = tot + es[j]
    acc = es[0] * xs[0]
    for j in range(1, k):
        acc = acc + es[j] * xs[j]
    out_ref[0] = acc / tot


def _p6_call(y5, s5, gb5, y3, s3, gb3, cnt):
    b, k, n, c = y5.shape
    return pl.pallas_call(
        functools.partial(_p6_body, k=k, cnt=cnt),
        grid=(b, n // _TN),
        in_specs=[
            pl.BlockSpec((1, k, _TN, c), lambda bb, i: (bb, 0, i, 0)),
            pl.BlockSpec((8, c), lambda bb, i: (0, 0)),
            pl.BlockSpec((8, c), lambda bb, i: (0, 0)),
            pl.BlockSpec((1, k, _TN, y3.shape[3]), lambda bb, i: (bb, 0, i, 0)),
            pl.BlockSpec((8, y3.shape[3]), lambda bb, i: (0, 0)),
            pl.BlockSpec((8, y3.shape[3]), lambda bb, i: (0, 0)),
        ],
        out_specs=pl.BlockSpec((1, _TN, y3.shape[3]), lambda bb, i: (bb, i, 0)),
        out_shape=jax.ShapeDtypeStruct((b, n, y3.shape[3]), F32),
    )(y5, s5, gb5, y3, s3, gb3)


def _q1_body(xyz_ref, g_ref, we_ref, ye_ref, se_ref, *, k):
    @pl.when(_first_step())
    def _():
        se_ref[...] = jnp.zeros_like(se_ref)

    q3 = xyz_ref[0][:, 0:3]
    ss = jnp.zeros((1, we_ref.shape[1]), F32)
    sq = jnp.zeros((1, we_ref.shape[1]), F32)
    for j in range(k):
        g = g_ref[0, j]
        gx = g[:, 64:67]
        dx = gx - q3
        euc = jnp.sqrt(jnp.sum(dx * dx, axis=1, keepdims=True) + 1e-20)
        xyzfeat = jnp.concatenate([q3, gx, dx, euc], axis=1)
        y = jnp.dot(xyzfeat, we_ref[...], preferred_element_type=F32)
        ye_ref[0, j] = y
        ss = ss + jnp.sum(y, axis=0, keepdims=True)
        sq = sq + jnp.sum(y * y, axis=0, keepdims=True)
    se_ref[0:1, :] = se_ref[0:1, :] + ss
    se_ref[1:2, :] = se_ref[1:2, :] + sq


def _q1_call(xyz8, g2, we):
    b, k, n, _ = g2.shape
    co = we.shape[1]
    return pl.pallas_call(
        functools.partial(_q1_body, k=k),
        grid=(b, n // _TN),
        in_specs=[
            pl.BlockSpec((1, _TN, 8), lambda bb, i: (bb, i, 0)),
            pl.BlockSpec((1, k, _TN, _TW), lambda bb, i: (bb, 0, i, 0)),
            pl.BlockSpec(we.shape, lambda bb, i: (0, 0)),
        ],
        out_specs=[
            pl.BlockSpec((1, k, _TN, co), lambda bb, i: (bb, 0, i, 0)),
            pl.BlockSpec((8, co), lambda bb, i: (0, 0)),
        ],
        out_shape=[
            jax.ShapeDtypeStruct((b, k, n, co), F32),
            jax.ShapeDtypeStruct((8, co), F32),
        ],
    )(xyz8, g2, we)


def _q2_body(ye_ref, se_ref, gbe_ref, wp_ref, g_ref, wa_ref, wb_ref, wc_ref,
             y6_ref, s6_ref, *, k, cnt):
    @pl.when(_first_step())
    def _():
        s6_ref[...] = jnp.zeros_like(s6_ref)

    esc, esh = _bn_fold(se_ref, gbe_ref, cnt)
    wp = wp_ref[0]
    wpy = jnp.dot(wp, wb_ref[...], preferred_element_type=F32)
    ss = jnp.zeros((1, wa_ref.shape[1]), F32)
    sq = jnp.zeros((1, wa_ref.shape[1]), F32)
    for j in range(k):
        xe = jnp.maximum(ye_ref[0, j] * esc + esh, 0.0)
        gp = g_ref[0, j][:, 0:64]
        y = (jnp.dot(xe, wa_ref[...], preferred_element_type=F32)
             + wpy
             + jnp.dot(gp, wc_ref[...], preferred_element_type=F32))
        y6_ref[0, j] = y
        ss = ss + jnp.sum(y, axis=0, keepdims=True)
        sq = sq + jnp.sum(y * y, axis=0, keepdims=True)
    s6_ref[0:1, :] = s6_ref[0:1, :] + ss
    s6_ref[1:2, :] = s6_ref[1:2, :] + sq


def _q2_call(ye2, se2, gbe2, wpoints, g2, wa, wb, wc, cnt):
    b, k, n, ce = ye2.shape
    co = wa.shape[1]
    return pl.pallas_call(
        functools.partial(_q2_body, k=k, cnt=cnt),
        grid=(b, n // _TN),
        in_specs=[
            pl.BlockSpec((1, k, _TN, ce), lambda bb, i: (bb, 0, i, 0)),
            pl.BlockSpec((8, ce), lambda bb, i: (0, 0)),
            pl.BlockSpec((8, ce), lambda bb, i: (0, 0)),
            pl.BlockSpec((1, _TN, wpoints.shape[2]), lambda bb, i: (bb, i, 0)),
            pl.BlockSpec((1, k, _TN, _TW), lambda bb, i: (bb, 0, i, 0)),
            pl.BlockSpec(wa.shape, lambda bb, i: (0, 0)),
            pl.BlockSpec(wb.shape, lambda bb, i: (0, 0)),
            pl.BlockSpec(wc.shape, lambda bb, i: (0, 0)),
        ],
        out_specs=[
            pl.BlockSpec((1, k, _TN, co), lambda bb, i: (bb, 0, i, 0)),
            pl.BlockSpec((8, co), lambda bb, i: (0, 0)),
        ],
        out_shape=[
            jax.ShapeDtypeStruct((b, k, n, co), F32),
            jax.ShapeDtypeStruct((8, co), F32),
        ],
    )(ye2, se2, gbe2, wpoints, g2, wa, wb, wc)


def _q4_body(y7_ref, s7_ref, gb7_ref, g_ref, out_ref, *, k, cnt):
    wsc, wsh = _bn_fold(s7_ref, gb7_ref, cnt)
    zs = [jnp.maximum(y7_ref[0, j] * wsc + wsh, 0.0) for j in range(k)]
    xs = [g_ref[0, j][:, 0:64] for j in range(k)]
    m = zs[0]
    for j in range(1, k):
        m = jnp.maximum(m, zs[j])
    es = [jnp.exp(z - m) for z in zs]
    tot = es[0]
    for j in range(1, k):
        tot = tot + es[j]
    acc = es[0] * xs[0]
    for j in range(1, k):
        acc = acc + es[j] * xs[j]
    out_ref[0] = acc / tot


def _q4_call(y7, s7, gb7, g2, cnt):
    b, k, n, c = y7.shape
    return pl.pallas_call(
        functools.partial(_q4_body, k=k, cnt=cnt),
        grid=(b, n // _TN),
        in_specs=[
            pl.BlockSpec((1, k, _TN, c), lambda bb, i: (bb, 0, i, 0)),
            pl.BlockSpec((8, c), lambda bb, i: (0, 0)),
            pl.BlockSpec((8, c), lambda bb, i: (0, 0)),
            pl.BlockSpec((1, k, _TN, _TW), lambda bb, i: (bb, 0, i, 0)),
        ],
        out_specs=pl.BlockSpec((1, _TN, c), lambda bb, i: (bb, i, 0)),
        out_shape=jax.ShapeDtypeStruct((b, n, c), F32),
    )(y7, s7, gb7, g2)


def _gb(gb_pair):
    gamma, beta = gb_pair
    return jnp.concatenate(
        [gamma[None], beta[None], jnp.zeros((6,) + gamma.shape, F32)], axis=0)


def kernel(warped_xyz, f2_xyz, warped_points, f2_points, params):
    b, n, _ = warped_xyz.shape
    m = f2_xyz.shape[1]

    pad_n = jnp.zeros((b, n, 5), F32)
    pad_m = jnp.zeros((b, m, 5), F32)
    wxyz8 = jnp.concatenate([warped_xyz, pad_n], axis=-1)
    fxyz8 = jnp.concatenate([f2_xyz, pad_m], axis=-1)
    fxyzT = jnp.swapaxes(fxyz8, 1, 2)
    wxyzT = jnp.swapaxes(wxyz8, 1, 2)

    idx1, idx2, table1 = _knn_call(wxyz8, fxyzT, wxyzT, f2_points, fxyz8)
    idxq = jnp.swapaxes(idx1, 1, 2).reshape(-1)
    g1 = _gather_rows(table1, idxq).reshape(b, _K1, n, _TW)

    w1 = params['conv1'][0][0]
    y1, ye, s1, se = _p1_call(wxyz8, warped_points, g1,
                              w1[0:10], w1[10:74], w1[74:138],
                              params['enc1'][0])
    cnt1 = float(b * n * _K1)
    y2, s2 = _layer_call(y1, s1, _gb(params['bn1s'][0]),
                         params['conv1'][1][0], cnt1)
    y3, s3 = _layer_call(y2, s2, _gb(params['bn1s'][1]),
                         params['conv1'][2][0], cnt1)
    gb3 = _gb(params['bn1s'][2])
    gbe = _gb(params['bn_e1'])
    w4 = params['conv2'][0][0]
    y4, s4 = _p4_call(ye, se, gbe, y3, s3, gb3, w4[0:64], w4[64:128], cnt1)
    y5, s5 = _layer_call(y4, s4, _gb(params['bn2s'][0]),
                         params['conv2'][1][0], cnt1)
    feat1 = _p6_call(y5, s5, _gb(params['bn2s'][1]), y3, s3, gb3, cnt1)

    idxp = jnp.swapaxes(idx2, 1, 2).reshape(-1)
    table2 = jnp.concatenate(
        [feat1, wxyz8, jnp.zeros((b, n, _TW - 72), F32)],
        axis=-1).reshape(b * n, _TW)
    g2 = _gather_rows(table2, idxp).reshape(b, _K2, n, _TW)

    ye2, se2 = _q1_call(wxyz8, g2, params['enc2'][0])
    cnt2 = float(b * n * _K2)
    w6 = params['conv3'][0][0]
    y6, s6 = _q2_call(ye2, se2, _gb(params['bn_e2']), warped_points, g2,
                      w6[0:64], w6[64:128], w6[128:192], cnt2)
    y7, s7 = _layer_call(y6, s6, _gb(params['bn2s'][0]),
                         params['conv3'][1][0], cnt2)
    return _q4_call(y7, s7, _gb(params['bn2s'][1]), g2, cnt2)


# speedup vs baseline: 28.3018x; 1.0359x over previous
import functools

import jax
import jax.numpy as jnp
from jax import lax
from jax.experimental import pallas as pl
from jax.experimental.pallas import tpu as pltpu
from jax.experimental.pallas import tpu_sc as plsc

F32 = jnp.float32
_TN = 1024
_K1 = 6
_K2 = 4
_TW = 128
_NW = 32
_CHUNK = 128


def _first_step():
    return (pl.program_id(0) == 0) & (pl.program_id(1) == 0)


_TNK = 512


def _topk_idx(q, xt, k, m):
    qsq = jnp.sum(q * q, axis=1, keepdims=True)
    ones_q = jnp.ones_like(qsq)
    qq = jnp.concatenate([-2.0 * q[:, 0:3], qsq, ones_q], axis=1)
    xsq = jnp.sum(xt * xt, axis=0, keepdims=True)
    xtt = jnp.concatenate([xt[0:3], jnp.ones_like(xsq), xsq], axis=0)
    d = jnp.dot(qq, xtt, preferred_element_type=F32)
    iota = lax.broadcasted_iota(jnp.int32, d.shape, 1)
    cols = []
    for _ in range(k):
        am = jnp.argmin(d, axis=1)[:, None]
        cols.append(am)
        d = jnp.where(iota == am, jnp.inf, d)
    return jnp.concatenate(cols, axis=1) + pl.program_id(0) * m


def _knn_body(q_ref, fxt_ref, wxt_ref, fp_ref, fx_ref,
              idx1_ref, idx2_ref, tab_ref, *, m):
    q = q_ref[0]
    idx1_ref[0] = _topk_idx(q, fxt_ref[0], _K1, m)
    idx2_ref[0] = _topk_idx(q, wxt_ref[0], _K2, m)
    fp = fp_ref[0]
    fx = fx_ref[0]
    pad = jnp.zeros((fp.shape[0], _TW - 72), F32)
    tab_ref[...] = jnp.concatenate([fp, fx, pad], axis=1)


def _knn_call(wxyz8, fxyzT, wxyzT, f2_points, fxyz8):
    b, n, _ = wxyz8.shape
    m = fxyzT.shape[2]
    nt = n // _TNK
    return pl.pallas_call(
        functools.partial(_knn_body, m=m),
        grid=(b, nt),
        in_specs=[
            pl.BlockSpec((1, _TNK, 8), lambda bb, i: (bb, i, 0)),
            pl.BlockSpec((1, 8, m), lambda bb, i: (bb, 0, 0)),
            pl.BlockSpec((1, 8, m), lambda bb, i: (bb, 0, 0)),
            pl.BlockSpec((1, _TNK, 64), lambda bb, i: (bb, i, 0)),
            pl.BlockSpec((1, _TNK, 8), lambda bb, i: (bb, i, 0)),
        ],
        out_specs=[
            pl.BlockSpec((1, _TNK, _K1), lambda bb, i: (bb, i, 0)),
            pl.BlockSpec((1, _TNK, _K2), lambda bb, i: (bb, i, 0)),
            pl.BlockSpec((_TNK, _TW), lambda bb, i, _nt=nt: (bb * _nt + i, 0)),
        ],
        out_shape=[
            jax.ShapeDtypeStruct((b, n, _K1), jnp.int32),
            jax.ShapeDtypeStruct((b, n, _K2), jnp.int32),
            jax.ShapeDtypeStruct((b * m, _TW), F32),
        ],
    )(wxyz8, fxyzT, wxyzT, f2_points, fxyz8)


def _gather_rows(table, idx):
    rows = idx.shape[0]
    per_w = rows // _NW
    chunks = per_w // _CHUNK
    idx3 = idx.reshape(_NW, chunks, _CHUNK)
    mesh = plsc.VectorSubcoreMesh(core_axis_name="c", subcore_axis_name="s",
                                  num_cores=2)

    @functools.partial(
        pl.kernel, mesh=mesh,
        out_type=jax.ShapeDtypeStruct((rows, _TW), F32),
        scratch_types=[
            pltpu.VMEM((chunks, _CHUNK), jnp.int32),
            pltpu.VMEM((2, _CHUNK, _TW), F32),
            pltpu.SemaphoreType.DMA,
            pltpu.SemaphoreType.DMA,
        ],
    )
    def gk(table_hbm, idx_hbm, out_hbm, idx_v, rows_v, sem0, sem1):
        wid = lax.axis_index("s") * 2 + lax.axis_index("c")
        base = wid * per_w
        pltpu.sync_copy(idx_hbm.at[wid], idx_v)
        sems = (sem0, sem1)
        prev = None
        for j in range(chunks):
            cp = pltpu.async_copy(table_hbm.at[idx_v.at[j]],
                                  rows_v.at[j % 2], sems[j % 2])
            if prev is not None:
                pj, pcp = prev
                pcp.wait()
                pltpu.sync_copy(
                    rows_v.at[pj % 2],
                    out_hbm.at[pl.ds(base + pj * _CHUNK, _CHUNK)])
            prev = (j, cp)
        pj, pcp = prev
        pcp.wait()
        pltpu.sync_copy(rows_v.at[pj % 2],
                        out_hbm.at[pl.ds(base + pj * _CHUNK, _CHUNK)])

    return gk(table, idx3)


def _p1_body(xyz_ref, wp_ref, g_ref, wq_ref, wx_ref, w9_ref, w1b_ref,
             w1c_ref, vq_ref, vx_ref, v9_ref, y1_ref, ye_ref, s1_ref, se_ref):
    @pl.when(_first_step())
    def _():
        s1_ref[...] = jnp.zeros_like(s1_ref)
        se_ref[...] = jnp.zeros_like(se_ref)

    q3 = xyz_ref[0][:, 0:3]
    wp = wp_ref[0]
    base = (jnp.dot(q3, wq_ref[...], preferred_element_type=F32)
            + jnp.dot(wp, w1b_ref[...], preferred_element_type=F32))
    base_e = jnp.dot(q3, vq_ref[...], preferred_element_type=F32)
    s1s = jnp.zeros((1, y1_ref.shape[-1]), F32)
    s1q = jnp.zeros((1, y1_ref.shape[-1]), F32)
    ses = jnp.zeros((1, ye_ref.shape[-1]), F32)
    seq = jnp.zeros((1, ye_ref.shape[-1]), F32)
    for j in range(_K1):
        g = g_ref[0, j]
        gp = g[:, 0:64]
        gx = g[:, 64:67]
        dx = gx - q3
        euc = jnp.sqrt(jnp.sum(dx * dx, axis=1, keepdims=True) + 1e-20)
        y = (base
             + jnp.dot(gx, wx_ref[...], preferred_element_type=F32)
             + jnp.dot(euc, w9_ref[...], preferred_element_type=F32)
             + jnp.dot(gp, w1c_ref[...], preferred_element_type=F32))
        ye = (base_e
              + jnp.dot(gx, vx_ref[...], preferred_element_type=F32)
              + jnp.dot(euc, v9_ref[...], preferred_element_type=F32))
        y1_ref[0, j] = y
        ye_ref[0, j] = ye
        s1s = s1s + jnp.sum(y, axis=0, keepdims=True)
        s1q = s1q + jnp.sum(y * y, axis=0, keepdims=True)
        ses = ses + jnp.sum(ye, axis=0, keepdims=True)
        seq = seq + jnp.sum(ye * ye, axis=0, keepdims=True)
    s1_ref[0:1, :] = s1_ref[0:1, :] + s1s
    s1_ref[1:2, :] = s1_ref[1:2, :] + s1q
    se_ref[0:1, :] = se_ref[0:1, :] + ses
    se_ref[1:2, :] = se_ref[1:2, :] + seq


def _split_xyz_weights(w10):
    wq = w10[0:3] - w10[6:9]
    wx = w10[3:6] + w10[6:9]
    return wq, wx, w10[9:10]


def _p1_call(xyz8, wpoints, g1, w1a, w1b, w1c, we):
    b, k, n, _ = g1.shape
    co, ce = w1a.shape[1], we.shape[1]
    wq, wx, w9 = _split_xyz_weights(w1a)
    vq, vx, v9 = _split_xyz_weights(we)
    small = [wq, wx, w9, w1b, w1c, vq, vx, v9]
    return pl.pallas_call(
        _p1_body,
        grid=(b, n // _TN),
        in_specs=[
            pl.BlockSpec((1, _TN, 8), lambda bb, i: (bb, i, 0)),
            pl.BlockSpec((1, _TN, wpoints.shape[2]), lambda bb, i: (bb, i, 0)),
            pl.BlockSpec((1, k, _TN, _TW), lambda bb, i: (bb, 0, i, 0)),
        ] + [pl.BlockSpec(w.shape, lambda bb, i: (0, 0)) for w in small],
        out_specs=[
            pl.BlockSpec((1, k, _TN, co), lambda bb, i: (bb, 0, i, 0)),
            pl.BlockSpec((1, k, _TN, ce), lambda bb, i: (bb, 0, i, 0)),
            pl.BlockSpec((8, co), lambda bb, i: (0, 0)),
            pl.BlockSpec((8, ce), lambda bb, i: (0, 0)),
        ],
        out_shape=[
            jax.ShapeDtypeStruct((b, k, n, co), F32),
            jax.ShapeDtypeStruct((b, k, n, ce), F32),
            jax.ShapeDtypeStruct((8, co), F32),
            jax.ShapeDtypeStruct((8, ce), F32),
        ],
    )(xyz8, wpoints, g1, *small)


def _bn_fold(s_ref, gb_ref, cnt):
    mean = s_ref[0:1, :] / cnt
    var = s_ref[1:2, :] / cnt - mean * mean
    sc = gb_ref[0:1, :] / jnp.sqrt(var + 1e-5)
    sh = gb_ref[1:2, :] - mean * sc
    return sc, sh


def _layer_body(yin_ref, sin_ref, gb_ref, w_ref, yout_ref, s_ref, *, k, cnt):
    @pl.when(_first_step())
    def _():
        s_ref[...] = jnp.zeros_like(s_ref)

    sc, sh = _bn_fold(sin_ref, gb_ref, cnt)
    ss = jnp.zeros((1, w_ref.shape[1]), F32)
    sq = jnp.zeros((1, w_ref.shape[1]), F32)
    for j in range(k):
        x = jnp.maximum(yin_ref[0, j] * sc + sh, 0.0)
        y = jnp.dot(x, w_ref[...], preferred_element_type=F32)
        yout_ref[0, j] = y
        ss = ss + jnp.sum(y, axis=0, keepdims=True)
        sq = sq + jnp.sum(y * y, axis=0, keepdims=True)
    s_ref[0:1, :] = s_ref[0:1, :] + ss
    s_ref[1:2, :] = s_ref[1:2, :] + sq


def _layer_call(yin, sin, gb, w, cnt):
    b, k, n, cin = yin.shape
    co = w.shape[1]
    return pl.pallas_call(
        functools.partial(_layer_body, k=k, cnt=cnt),
        grid=(b, n // _TN),
        in_specs=[
            pl.BlockSpec((1, k, _TN, cin), lambda bb, i: (bb, 0, i, 0)),
            pl.BlockSpec((8, cin), lambda bb, i: (0, 0)),
            pl.BlockSpec((8, cin), lambda bb, i: (0, 0)),
            pl.BlockSpec(w.shape, lambda bb, i: (0, 0)),
        ],
        out_specs=[
            pl.BlockSpec((1, k, _TN, co), lambda bb, i: (bb, 0, i, 0)),
            pl.BlockSpec((8, co), lambda bb, i: (0, 0)),
        ],
        out_shape=[
            jax.ShapeDtypeStruct((b, k, n, co), F32),
            jax.ShapeDtypeStruct((8, co), F32),
        ],
    )(yin, sin, gb, w)


def _p4_body(ye_ref, se_ref, gbe_ref, y3_ref, s3_ref, gb3_ref, wa_ref, wb_ref,
             y4_ref, s4_ref, *, k, cnt):
    @pl.when(_first_step())
    def _():
        s4_ref[...] = jnp.zeros_like(s4_ref)

    esc, esh = _bn_fold(se_ref, gbe_ref, cnt)
    xsc, xsh = _bn_fold(s3_ref, gb3_ref, cnt)
    ss = jnp.zeros((1, wa_ref.shape[1]), F32)
    sq = jnp.zeros((1, wa_ref.shape[1]), F32)
    for j in range(k):
        xe = jnp.maximum(ye_ref[0, j] * esc + esh, 0.0)
        x3 = jnp.maximum(y3_ref[0, j] * xsc + xsh, 0.0)
        y = (jnp.dot(xe, wa_ref[...], preferred_element_type=F32)
             + jnp.dot(x3, wb_ref[...], preferred_element_type=F32))
        y4_ref[0, j] = y
        ss = ss + jnp.sum(y, axis=0, keepdims=True)
        sq = sq + jnp.sum(y * y, axis=0, keepdims=True)
    s4_ref[0:1, :] = s4_ref[0:1, :] + ss
    s4_ref[1:2, :] = s4_ref[1:2, :] + sq


def _p4_call(ye, se, gbe, y3, s3, gb3, wa, wb, cnt):
    b, k, n, ce = ye.shape
    co = wa.shape[1]
    return pl.pallas_call(
        functools.partial(_p4_body, k=k, cnt=cnt),
        grid=(b, n // _TN),
        in_specs=[
            pl.BlockSpec((1, k, _TN, ce), lambda bb, i: (bb, 0, i, 0)),
            pl.BlockSpec((8, ce), lambda bb, i: (0, 0)),
            pl.BlockSpec((8, ce), lambda bb, i: (0, 0)),
            pl.BlockSpec((1, k, _TN, y3.shape[3]), lambda bb, i: (bb, 0, i, 0)),
            pl.BlockSpec((8, y3.shape[3]), lambda bb, i: (0, 0)),
            pl.BlockSpec((8, y3.shape[3]), lambda bb, i: (0, 0)),
            pl.BlockSpec(wa.shape, lambda bb, i: (0, 0)),
            pl.BlockSpec(wb.shape, lambda bb, i: (0, 0)),
        ],
        out_specs=[
            pl.BlockSpec((1, k, _TN, co), lambda bb, i: (bb, 0, i, 0)),
            pl.BlockSpec((8, co), lambda bb, i: (0, 0)),
        ],
        out_shape=[
            jax.ShapeDtypeStruct((b, k, n, co), F32),
            jax.ShapeDtypeStruct((8, co), F32),
        ],
    )(ye, se, gbe, y3, s3, gb3, wa, wb)


def _p6_body(y5_ref, s5_ref, gb5_ref, y3_ref, s3_ref, gb3_ref, out_ref,
             *, k, cnt):
    wsc, wsh = _bn_fold(s5_ref, gb5_ref, cnt)
    xsc, xsh = _bn_fold(s3_ref, gb3_ref, cnt)
    zs = [jnp.maximum(y5_ref[0, j] * wsc + wsh, 0.0) for j in range(k)]
    xs = [jnp.maximum(y3_ref[0, j] * xsc + xsh, 0.0) for j in range(k)]
    m = zs[0]
    for j in range(1, k):
        m = jnp.maximum(m, zs[j])
    es = [jnp.exp(z - m) for z in zs]
    tot = es[0]
    for j in range(1, k):
        tot = tot + es[j]
    acc = es[0] * xs[0]
    for j in range(1, k):
        acc = acc + es[j] * xs[j]
    out_ref[0] = acc / tot


def _p6_call(y5, s5, gb5, y3, s3, gb3, cnt):
    b, k, n, c = y5.shape
    return pl.pallas_call(
        functools.partial(_p6_body, k=k, cnt=cnt),
        grid=(b, n // _TN),
        in_specs=[
            pl.BlockSpec((1, k, _TN, c), lambda bb, i: (bb, 0, i, 0)),
            pl.BlockSpec((8, c), lambda bb, i: (0, 0)),
            pl.BlockSpec((8, c), lambda bb, i: (0, 0)),
            pl.BlockSpec((1, k, _TN, y3.shape[3]), lambda bb, i: (bb, 0, i, 0)),
            pl.BlockSpec((8, y3.shape[3]), lambda bb, i: (0, 0)),
            pl.BlockSpec((8, y3.shape[3]), lambda bb, i: (0, 0)),
        ],
        out_specs=pl.BlockSpec((1, _TN, y3.shape[3]), lambda bb, i: (bb, i, 0)),
        out_shape=jax.ShapeDtypeStruct((b, n, y3.shape[3]), F32),
    )(y5, s5, gb5, y3, s3, gb3)


def _q1_body(xyz_ref, g_ref, vq_ref, vx_ref, v9_ref, ye_ref, se_ref, *, k):
    @pl.when(_first_step())
    def _():
        se_ref[...] = jnp.zeros_like(se_ref)

    q3 = xyz_ref[0][:, 0:3]
    base_e = jnp.dot(q3, vq_ref[...], preferred_element_type=F32)
    ss = jnp.zeros((1, v9_ref.shape[1]), F32)
    sq = jnp.zeros((1, v9_ref.shape[1]), F32)
    for j in range(k):
        g = g_ref[0, j]
        gx = g[:, 64:67]
        dx = gx - q3
        euc = jnp.sqrt(jnp.sum(dx * dx, axis=1, keepdims=True) + 1e-20)
        y = (base_e
             + jnp.dot(gx, vx_ref[...], preferred_element_type=F32)
             + jnp.dot(euc, v9_ref[...], preferred_element_type=F32))
        ye_ref[0, j] = y
        ss = ss + jnp.sum(y, axis=0, keepdims=True)
        sq = sq + jnp.sum(y * y, axis=0, keepdims=True)
    se_ref[0:1, :] = se_ref[0:1, :] + ss
    se_ref[1:2, :] = se_ref[1:2, :] + sq


def _q1_call(xyz8, g2, we):
    b, k, n, _ = g2.shape
    co = we.shape[1]
    vq, vx, v9 = _split_xyz_weights(we)
    return pl.pallas_call(
        functools.partial(_q1_body, k=k),
        grid=(b, n // _TN),
        in_specs=[
            pl.BlockSpec((1, _TN, 8), lambda bb, i: (bb, i, 0)),
            pl.BlockSpec((1, k, _TN, _TW), lambda bb, i: (bb, 0, i, 0)),
            pl.BlockSpec(vq.shape, lambda bb, i: (0, 0)),
            pl.BlockSpec(vx.shape, lambda bb, i: (0, 0)),
            pl.BlockSpec(v9.shape, lambda bb, i: (0, 0)),
        ],
        out_specs=[
            pl.BlockSpec((1, k, _TN, co), lambda bb, i: (bb, 0, i, 0)),
            pl.BlockSpec((8, co), lambda bb, i: (0, 0)),
        ],
        out_shape=[
            jax.ShapeDtypeStruct((b, k, n, co), F32),
            jax.ShapeDtypeStruct((8, co), F32),
        ],
    )(xyz8, g2, vq, vx, v9)


def _q2_body(ye_ref, se_ref, gbe_ref, wp_ref, g_ref, wa_ref, wb_ref, wc_ref,
             y6_ref, s6_ref, *, k, cnt):
    @pl.when(_first_step())
    def _():
        s6_ref[...] = jnp.zeros_like(s6_ref)

    esc, esh = _bn_fold(se_ref, gbe_ref, cnt)
    wp = wp_ref[0]
    wpy = jnp.dot(wp, wb_ref[...], preferred_element_type=F32)
    ss = jnp.zeros((1, wa_ref.shape[1]), F32)
    sq = jnp.zeros((1, wa_ref.shape[1]), F32)
    for j in range(k):
        xe = jnp.maximum(ye_ref[0, j] * esc + esh, 0.0)
        gp = g_ref[0, j][:, 0:64]
        y = (jnp.dot(xe, wa_ref[...], preferred_element_type=F32)
             + wpy
             + jnp.dot(gp, wc_ref[...], preferred_element_type=F32))
        y6_ref[0, j] = y
        ss = ss + jnp.sum(y, axis=0, keepdims=True)
        sq = sq + jnp.sum(y * y, axis=0, keepdims=True)
    s6_ref[0:1, :] = s6_ref[0:1, :] + ss
    s6_ref[1:2, :] = s6_ref[1:2, :] + sq


def _q2_call(ye2, se2, gbe2, wpoints, g2, wa, wb, wc, cnt):
    b, k, n, ce = ye2.shape
    co = wa.shape[1]
    return pl.pallas_call(
        functools.partial(_q2_body, k=k, cnt=cnt),
        grid=(b, n // _TN),
        in_specs=[
            pl.BlockSpec((1, k, _TN, ce), lambda bb, i: (bb, 0, i, 0)),
            pl.BlockSpec((8, ce), lambda bb, i: (0, 0)),
            pl.BlockSpec((8, ce), lambda bb, i: (0, 0)),
            pl.BlockSpec((1, _TN, wpoints.shape[2]), lambda bb, i: (bb, i, 0)),
            pl.BlockSpec((1, k, _TN, _TW), lambda bb, i: (bb, 0, i, 0)),
            pl.BlockSpec(wa.shape, lambda bb, i: (0, 0)),
            pl.BlockSpec(wb.shape, lambda bb, i: (0, 0)),
            pl.BlockSpec(wc.shape, lambda bb, i: (0, 0)),
        ],
        out_specs=[
            pl.BlockSpec((1, k, _TN, co), lambda bb, i: (bb, 0, i, 0)),
            pl.BlockSpec((8, co), lambda bb, i: (0, 0)),
        ],
        out_shape=[
            jax.ShapeDtypeStruct((b, k, n, co), F32),
            jax.ShapeDtypeStruct((8, co), F32),
        ],
    )(ye2, se2, gbe2, wpoints, g2, wa, wb, wc)


def _q4_body(y7_ref, s7_ref, gb7_ref, g_ref, out_ref, *, k, cnt):
    wsc, wsh = _bn_fold(s7_ref, gb7_ref, cnt)
    zs = [jnp.maximum(y7_ref[0, j] * wsc + wsh, 0.0) for j in range(k)]
    xs = [g_ref[0, j][:, 0:64] for j in range(k)]
    m = zs[0]
    for j in range(1, k):
        m = jnp.maximum(m, zs[j])
    es = [jnp.exp(z - m) for z in zs]
    tot = es[0]
    for j in range(1, k):
        tot = tot + es[j]
    acc = es[0] * xs[0]
    for j in range(1, k):
        acc = acc + es[j] * xs[j]
    out_ref[0] = acc / tot


def _q4_call(y7, s7, gb7, g2, cnt):
    b, k, n, c = y7.shape
    return pl.pallas_call(
        functools.partial(_q4_body, k=k, cnt=cnt),
        grid=(b, n // _TN),
        in_specs=[
            pl.BlockSpec((1, k, _TN, c), lambda bb, i: (bb, 0, i, 0)),
            pl.BlockSpec((8, c), lambda bb, i: (0, 0)),
            pl.BlockSpec((8, c), lambda bb, i: (0, 0)),
            pl.BlockSpec((1, k, _TN, _TW), lambda bb, i: (bb, 0, i, 0)),
        ],
        out_specs=pl.BlockSpec((1, _TN, c), lambda bb, i: (bb, i, 0)),
        out_shape=jax.ShapeDtypeStruct((b, n, c), F32),
    )(y7, s7, gb7, g2)


def _gb(gb_pair):
    gamma, beta = gb_pair
    return jnp.concatenate(
        [gamma[None], beta[None], jnp.zeros((6,) + gamma.shape, F32)], axis=0)


def kernel(warped_xyz, f2_xyz, warped_points, f2_points, params):
    b, n, _ = warped_xyz.shape
    m = f2_xyz.shape[1]

    pad_n = jnp.zeros((b, n, 5), F32)
    pad_m = jnp.zeros((b, m, 5), F32)
    wxyz8 = jnp.concatenate([warped_xyz, pad_n], axis=-1)
    fxyz8 = jnp.concatenate([f2_xyz, pad_m], axis=-1)
    fxyzT = jnp.swapaxes(fxyz8, 1, 2)
    wxyzT = jnp.swapaxes(wxyz8, 1, 2)

    idx1, idx2, table1 = _knn_call(wxyz8, fxyzT, wxyzT, f2_points, fxyz8)
    idxq = jnp.swapaxes(idx1, 1, 2).reshape(-1)
    g1 = _gather_rows(table1, idxq).reshape(b, _K1, n, _TW)

    w1 = params['conv1'][0][0]
    y1, ye, s1, se = _p1_call(wxyz8, warped_points, g1,
                              w1[0:10], w1[10:74], w1[74:138],
                              params['enc1'][0])
    cnt1 = float(b * n * _K1)
    y2, s2 = _layer_call(y1, s1, _gb(params['bn1s'][0]),
                         params['conv1'][1][0], cnt1)
    y3, s3 = _layer_call(y2, s2, _gb(params['bn1s'][1]),
                         params['conv1'][2][0], cnt1)
    gb3 = _gb(params['bn1s'][2])
    gbe = _gb(params['bn_e1'])
    w4 = params['conv2'][0][0]
    y4, s4 = _p4_call(ye, se, gbe, y3, s3, gb3, w4[0:64], w4[64:128], cnt1)
    y5, s5 = _layer_call(y4, s4, _gb(params['bn2s'][0]),
                         params['conv2'][1][0], cnt1)
    feat1 = _p6_call(y5, s5, _gb(params['bn2s'][1]), y3, s3, gb3, cnt1)

    idxp = jnp.swapaxes(idx2, 1, 2).reshape(-1)
    table2 = jnp.concatenate(
        [feat1, wxyz8, jnp.zeros((b, n, _TW - 72), F32)],
        axis=-1).reshape(b * n, _TW)
    g2 = _gather_rows(table2, idxp).reshape(b, _K2, n, _TW)

    ye2, se2 = _q1_call(wxyz8, g2, params['enc2'][0])
    cnt2 = float(b * n * _K2)
    w6 = params['conv3'][0][0]
    y6, s6 = _q2_call(ye2, se2, _gb(params['bn_e2']), warped_points, g2,
                      w6[0:64], w6[64:128], w6[128:192], cnt2)
    y7, s7 = _layer_call(y6, s6, _gb(params['bn2s'][0]),
                         params['conv3'][1][0], cnt2)
    return _q4_call(y7, s7, _gb(params['bn2s'][1]), g2, cnt2)
